# Initial kernel scaffold; baseline (speedup 1.0000x reference)
import jax
import jax.numpy as jnp
from jax.experimental import pallas as pl


def kernel(constraint_features, edge_indices, edge_features, variable_features, params):
    raise NotImplementedError("write your pallas kernel here")


# speedup vs baseline: 1.9756x; 1.9756x over previous
import functools

import jax
import jax.numpy as jnp
from jax import lax
from jax.experimental import pallas as pl
from jax.experimental.pallas import tpu as pltpu
from jax.experimental.pallas import tpu_sc as plsc

EMB = 64
LANES = 16
IDXW = 128
CHUNK = 512
GROUP = 1024
TRASH = 128

_EPS = 1e-5


def _ln_block(x, g, b):
    mu = jnp.mean(x, axis=-1, keepdims=True)
    var = jnp.mean((x - mu) * (x - mu), axis=-1, keepdims=True)
    return (x - mu) * lax.rsqrt(var + _EPS) * g + b


def _embed_body(x_ref, lng_ref, lnb_ref, w1_ref, b1_ref, w2_ref, b2_ref, o_ref):
    x = _ln_block(x_ref[...], lng_ref[...], lnb_ref[...])
    h = jnp.maximum(
        jnp.dot(x, w1_ref[...], preferred_element_type=jnp.float32) + b1_ref[...], 0.0)
    o_ref[...] = jnp.maximum(
        jnp.dot(h, w2_ref[...], preferred_element_type=jnp.float32) + b2_ref[...], 0.0)


def _tc_embed(x, lng, lnb, w1t, b1, w2t, b2, blk):
    n, f = x.shape
    grid = (n // blk,)
    full = lambda a: pl.BlockSpec(a.shape, lambda i: (0,) * a.ndim)
    return pl.pallas_call(
        _embed_body,
        grid=grid,
        in_specs=[
            pl.BlockSpec((blk, f), lambda i: (i, 0)),
            full(lng), full(lnb), full(w1t), full(b1), full(w2t), full(b2),
        ],
        out_specs=pl.BlockSpec((blk, EMB), lambda i: (i, 0)),
        out_shape=jax.ShapeDtypeStruct((n, EMB), jnp.float32),
    )(x, lng, lnb, w1t, b1, w2t, b2)


def _nodeprep_body(r_ref, l_ref, wlt_ref, ba_ref, wrt_ref, a_ref, b_ref):
    a_ref[...] = jnp.dot(
        r_ref[...], wlt_ref[...], preferred_element_type=jnp.float32) + ba_ref[...]
    b_ref[...] = jnp.dot(
        l_ref[...], wrt_ref[...], preferred_element_type=jnp.float32)


def _tc_nodeprep(r_emb, l_emb, wlt, bias_a, wrt, blk):
    n = r_emb.shape[0]
    grid = (n // blk,)
    full = lambda a: pl.BlockSpec(a.shape, lambda i: (0,) * a.ndim)
    return pl.pallas_call(
        _nodeprep_body,
        grid=grid,
        in_specs=[
            pl.BlockSpec((blk, EMB), lambda i: (i, 0)),
            pl.BlockSpec((blk, EMB), lambda i: (i, 0)),
            full(wlt), full(bias_a), full(wrt),
        ],
        out_specs=[
            pl.BlockSpec((blk, EMB), lambda i: (i, 0)),
            pl.BlockSpec((blk, EMB), lambda i: (i, 0)),
        ],
        out_shape=[
            jax.ShapeDtypeStruct((n, EMB), jnp.float32),
            jax.ShapeDtypeStruct((n, EMB), jnp.float32),
        ],
    )(r_emb, l_emb, wlt, bias_a, wrt)


def _edge_body(m_ref, g_ref, b_ref, wft_ref, bf_ref, lo_ref, hi_ref):
    t = jnp.maximum(_ln_block(m_ref[...], g_ref[...], b_ref[...]), 0.0)
    mf = jnp.dot(t, wft_ref[...], preferred_element_type=jnp.float32) + bf_ref[...]
    lo_ref[...] = mf[:, : EMB // 2]
    hi_ref[...] = mf[:, EMB // 2 :]


def _tc_edge(m, g, b, wft, bf, blk):
    e = m.shape[0]
    grid = (e // blk,)
    full = lambda a: pl.BlockSpec(a.shape, lambda i: (0,) * a.ndim)
    return pl.pallas_call(
        _edge_body,
        grid=grid,
        in_specs=[
            pl.BlockSpec((blk, EMB), lambda i: (i, 0)),
            full(g), full(b), full(wft), full(bf),
        ],
        out_specs=[
            pl.BlockSpec((blk, EMB // 2), lambda i: (i, 0)),
            pl.BlockSpec((blk, EMB // 2), lambda i: (i, 0)),
        ],
        out_shape=[
            jax.ShapeDtypeStruct((e, EMB // 2), jnp.float32),
            jax.ShapeDtypeStruct((e, EMB // 2), jnp.float32),
        ],
    )(m, g, b, wft, bf)


def _post_body(alo_ref, ahi_ref, r_ref, png_ref, pnb_ref, wo1t_ref, bo1_ref,
               wo2t_ref, bo2_ref, lng_ref, lnb_ref, o_ref):
    agg = jnp.concatenate([alo_ref[...], ahi_ref[...]], axis=-1)
    a = _ln_block(agg, png_ref[...], pnb_ref[...])
    h = jnp.concatenate([a, r_ref[...]], axis=-1)
    h = jnp.maximum(
        jnp.dot(h, wo1t_ref[...], preferred_element_type=jnp.float32) + bo1_ref[...], 0.0)
    x = jnp.dot(h, wo2t_ref[...], preferred_element_type=jnp.float32) + bo2_ref[...]
    o_ref[...] = _ln_block(x, lng_ref[...], lnb_ref[...])


def _tc_post(alo, ahi, r_emb, png, pnb, wo1t, bo1, wo2t, bo2, lng, lnb, blk):
    n = r_emb.shape[0]
    grid = (n // blk,)
    full = lambda a: pl.BlockSpec(a.shape, lambda i: (0,) * a.ndim)
    return pl.pallas_call(
        _post_body,
        grid=grid,
        in_specs=[
            pl.BlockSpec((blk, EMB // 2), lambda i: (i, 0)),
            pl.BlockSpec((blk, EMB // 2), lambda i: (i, 0)),
            pl.BlockSpec((blk, EMB), lambda i: (i, 0)),
            full(png), full(pnb), full(wo1t), full(bo1),
            full(wo2t), full(bo2), full(lng), full(lnb),
        ],
        out_specs=pl.BlockSpec((blk, EMB), lambda i: (i, 0)),
        out_shape=jax.ShapeDtypeStruct((n, EMB), jnp.float32),
    )(alo, ahi, r_emb, png, pnb, wo1t, bo1, wo2t, bo2, lng, lnb)


def _final_body(v_ref, w1t_ref, b1_ref, w2t_ref, o_ref):
    h = jnp.maximum(
        jnp.dot(v_ref[...], w1t_ref[...], preferred_element_type=jnp.float32)
        + b1_ref[...], 0.0)
    o_ref[...] = jnp.dot(h, w2t_ref[...], preferred_element_type=jnp.float32)


def _tc_final(v, w1t, b1, w2t, blk):
    n = v.shape[0]
    grid = (n // blk,)
    full = lambda a: pl.BlockSpec(a.shape, lambda i: (0,) * a.ndim)
    return pl.pallas_call(
        _final_body,
        grid=grid,
        in_specs=[
            pl.BlockSpec((blk, EMB), lambda i: (i, 0)),
            full(w1t), full(b1), full(w2t),
        ],
        out_specs=pl.BlockSpec((blk, 1), lambda i: (i, 0)),
        out_shape=jax.ShapeDtypeStruct((n, 1), jnp.float32),
    )(v, w1t, b1, w2t)


@functools.lru_cache(maxsize=None)
def _make_sc_gather(n_a, n_b, e_pad):
    info = plsc.get_sparse_core_info()
    ncores, nsub = info.num_cores, info.num_subcores
    nw = ncores * nsub
    per_w = e_pad // nw
    n_groups = per_w // GROUP
    gidx = GROUP // IDXW
    nidx = CHUNK // IDXW
    nsub_chunks = GROUP // CHUNK
    mesh = plsc.VectorSubcoreMesh(core_axis_name="c", subcore_axis_name="s")

    @functools.partial(
        pl.kernel,
        mesh=mesh,
        compiler_params=pltpu.CompilerParams(use_tc_tiling_on_sc=False),
        out_type=jax.ShapeDtypeStruct((e_pad, EMB), jnp.float32),
        scratch_types=[
            pltpu.VMEM((gidx, IDXW), jnp.int32),
            pltpu.VMEM((gidx, IDXW), jnp.int32),
            pltpu.VMEM((CHUNK, EMB), jnp.float32),
            pltpu.VMEM((CHUNK, EMB), jnp.float32),
            pltpu.SemaphoreType.DMA,
            pltpu.SemaphoreType.DMA,
        ],
    )
    def k(a_hbm, b_hbm, dst_hbm, src_hbm, m_hbm, dsti, srci, rowa, rowb, sema, semb):
        wid = lax.axis_index("s") * ncores + lax.axis_index("c")
        base0 = wid * per_w

        def body(j, carry):
            gbase = pl.multiple_of(base0 + j * GROUP, GROUP)
            row0 = pl.multiple_of(gbase // IDXW, gidx)
            pltpu.sync_copy(dst_hbm.at[pl.ds(row0, gidx)], dsti)
            pltpu.sync_copy(src_hbm.at[pl.ds(row0, gidx)], srci)
            for h in range(nsub_chunks):
                base = pl.multiple_of(gbase + h * CHUNK, CHUNK)
                descs = []
                for q in range(nidx):
                    r = h * nidx + q
                    descs.append(pltpu.async_copy(
                        a_hbm.at[dsti.at[r]], rowa.at[pl.ds(q * IDXW, IDXW)], sema))
                    descs.append(pltpu.async_copy(
                        b_hbm.at[srci.at[r]], rowb.at[pl.ds(q * IDXW, IDXW)], semb))
                for d in descs:
                    d.wait()

                def add_row(i, c):
                    for c4 in range(EMB // LANES):
                        sl = pl.ds(c4 * LANES, LANES)
                        rowa[i, sl] = rowa[i, sl] + rowb[i, sl]
                    return c

                lax.fori_loop(0, CHUNK, add_row, 0)
                pltpu.sync_copy(rowa, m_hbm.at[pl.ds(base, CHUNK)])
            return carry

        lax.fori_loop(0, n_groups, body, 0)

    return k


@functools.lru_cache(maxsize=None)
def _make_sc_scatter(n_nodes, e_pad):
    info = plsc.get_sparse_core_info()
    ncores, nsub = info.num_cores, info.num_subcores
    half = EMB // 2
    per_tile_e = e_pad // nsub
    n_groups = per_tile_e // GROUP
    gidx = GROUP // IDXW
    nidx = CHUNK // IDXW
    nsub_chunks = GROUP // CHUNK
    unit = 400
    n_units = n_nodes // unit
    units_per_tile = (n_units + nsub - 1) // nsub
    mesh = plsc.VectorSubcoreMesh(core_axis_name="c", subcore_axis_name="s")

    out_sd = jax.ShapeDtypeStruct((n_nodes, half), jnp.float32)

    @functools.partial(
        pl.kernel,
        mesh=mesh,
        compiler_params=pltpu.CompilerParams(use_tc_tiling_on_sc=False),
        out_type=(out_sd, out_sd),
        scratch_types=[
            pltpu.VMEM((gidx, IDXW), jnp.int32),
            pltpu.VMEM((CHUNK, half), jnp.float32),
            pltpu.VMEM((unit, half), jnp.float32),
            pltpu.MemorySpace.VMEM_SHARED((n_nodes + TRASH, half), jnp.float32),
        ],
    )
    def k(lo_hbm, hi_hbm, dst_hbm, out_lo, out_hi, dsti, rows, zbuf, agg_sh):
        core = lax.axis_index("c")
        sub = lax.axis_index("s")

        def zb(i, c):
            for c2 in range(half // LANES):
                zbuf[i, pl.ds(c2 * LANES, LANES)] = jnp.zeros((LANES,), jnp.float32)
            return c

        lax.fori_loop(0, unit, zb, 0)

        def zz(i, c):
            u = i * nsub + sub

            @pl.when(u < n_units)
            def _():
                off = pl.multiple_of(u * unit, 8)
                pltpu.sync_copy(zbuf, agg_sh.at[pl.ds(off, unit)])

            return c

        lax.fori_loop(0, units_per_tile, zz, 0)

        @pl.when(sub == 0)
        def _():
            pltpu.sync_copy(zbuf.at[pl.ds(0, TRASH)],
                            agg_sh.at[pl.ds(n_nodes, TRASH)])

        plsc.subcore_barrier()

        base0 = sub * per_tile_e

        def body(j, carry):
            gbase = pl.multiple_of(base0 + j * GROUP, GROUP)
            row0 = pl.multiple_of(gbase // IDXW, gidx)
            pltpu.sync_copy(dst_hbm.at[pl.ds(row0, gidx)], dsti)
            for h in range(nsub_chunks):
                base = pl.multiple_of(gbase + h * CHUNK, CHUNK)

                @pl.when(core == 0)
                def _():
                    pltpu.sync_copy(lo_hbm.at[pl.ds(base, CHUNK)], rows)

                @pl.when(core == 1)
                def _():
                    pltpu.sync_copy(hi_hbm.at[pl.ds(base, CHUNK)], rows)

                for q in range(nidx):
                    pltpu.sync_copy(
                        rows.at[pl.ds(q * IDXW, IDXW)],
                        agg_sh.at[dsti.at[h * nidx + q]], add=True)
            return carry

        lax.fori_loop(0, n_groups, body, 0)
        plsc.subcore_barrier()

        def wb(i, c):
            u = i * nsub + sub

            @pl.when(u < n_units)
            def _():
                sl = pl.ds(pl.multiple_of(u * unit, 8), unit)

                @pl.when(core == 0)
                def _():
                    pltpu.sync_copy(agg_sh.at[sl], out_lo.at[sl])

                @pl.when(core == 1)
                def _():
                    pltpu.sync_copy(agg_sh.at[sl], out_hi.at[sl])

            return c

        lax.fori_loop(0, units_per_tile, wb, 0)

    return k


def _conv_dir(left_emb, right_emb, dst_g, src_g, dst_s, e_pad, pc, e_bias,
              out_lng, out_lnb, blk):
    n_r = right_emb.shape[0]
    n_l = left_emb.shape[0]
    const_e = e_bias * pc["We"][:, 0]
    bias_a = (pc["bl"] + const_e).reshape(1, EMB)
    a_nodes, b_nodes = _tc_nodeprep(
        right_emb, left_emb, pc["Wl"].T, bias_a, pc["Wr"].T, blk)
    m = _make_sc_gather(n_r, n_l, e_pad)(a_nodes, b_nodes, dst_g, src_g)
    mf_lo, mf_hi = _tc_edge(
        m, pc["ln_f_g"].reshape(1, EMB), pc["ln_f_b"].reshape(1, EMB),
        pc["Wf"].T, pc["bf"].reshape(1, EMB), 4096)
    agg_lo, agg_hi = _make_sc_scatter(n_r, e_pad)(mf_lo, mf_hi, dst_s)
    return _tc_post(
        agg_lo, agg_hi, right_emb,
        pc["ln_post_g"].reshape(1, EMB), pc["ln_post_b"].reshape(1, EMB),
        pc["Wo1"].T, pc["bo1"].reshape(1, EMB),
        pc["Wo2"].T, pc["bo2"].reshape(1, EMB),
        out_lng.reshape(1, EMB), out_lnb.reshape(1, EMB), blk)


def kernel(constraint_features, edge_indices, edge_features, variable_features, params):
    del edge_features
    p = params
    n_c = constraint_features.shape[0]
    n_v = variable_features.shape[0]
    e = edge_indices.shape[1]
    blk = 2000

    lane_chunk = 32 * GROUP
    e_pad = ((e + lane_chunk - 1) // lane_chunk) * lane_chunk

    ei0 = edge_indices[0].astype(jnp.int32)
    ei1 = edge_indices[1].astype(jnp.int32)
    pad = e_pad - e

    def pad_to(v, fill):
        return jnp.pad(v, (0, pad), constant_values=fill).reshape(e_pad // IDXW, IDXW)

    ei0_g = pad_to(ei0, 0)
    ei1_g = pad_to(ei1, 0)
    ei0_s = pad_to(ei0, n_c)
    ei1_s = pad_to(ei1, n_v)

    cemb = _tc_embed(
        constraint_features,
        p["ln_c_in_g"].reshape(1, -1), p["ln_c_in_b"].reshape(1, -1),
        p["Wc1"].T, p["bc1"].reshape(1, EMB), p["Wc2"].T, p["bc2"].reshape(1, EMB),
        blk)
    vemb = _tc_embed(
        variable_features,
        p["ln_v_in_g"].reshape(1, -1), p["ln_v_in_b"].reshape(1, -1),
        p["Wv1"].T, p["bv1"].reshape(1, EMB), p["Wv2"].T, p["bv2"].reshape(1, EMB),
        blk)

    e_bias = p["ln_e_b"][0]

    c2 = _conv_dir(vemb, cemb, ei0_g, ei1_g, ei0_s, e_pad, p["conv_vc"], e_bias,
                   p["ln_c_g"], p["ln_c_b"], blk)
    v2 = _conv_dir(c2, vemb, ei1_g, ei0_g, ei1_s, e_pad, p["conv_cv"], e_bias,
                   p["ln_v_g"], p["ln_v_b"], blk)

    out = _tc_final(v2, p["Wout1"].T, p["bout1"].reshape(1, EMB), p["Wout2"].T, blk)
    return out[:, 0]


# speedup vs baseline: 2.1488x; 1.0877x over previous
import functools

import jax
import jax.numpy as jnp
from jax import lax
from jax.experimental import pallas as pl
from jax.experimental.pallas import tpu as pltpu
from jax.experimental.pallas import tpu_sc as plsc

EMB = 64
LANES = 16
IDXW = 128
CHUNK = 512
GROUP = 1024
TRASH = 128

_EPS = 1e-5


def _ln_block(x, g, b):
    mu = jnp.mean(x, axis=-1, keepdims=True)
    var = jnp.mean((x - mu) * (x - mu), axis=-1, keepdims=True)
    return (x - mu) * lax.rsqrt(var + _EPS) * g + b


def _embed_body(nderiv, x_ref, lng_ref, lnb_ref, w1_ref, b1_ref, w2_ref, b2_ref,
                *rest):
    x = _ln_block(x_ref[...], lng_ref[...], lnb_ref[...])
    h = jnp.maximum(
        jnp.dot(x, w1_ref[...], preferred_element_type=jnp.float32) + b1_ref[...], 0.0)
    emb = jnp.maximum(
        jnp.dot(h, w2_ref[...], preferred_element_type=jnp.float32) + b2_ref[...], 0.0)
    wts = rest[: 2 * nderiv]
    outs = rest[2 * nderiv :]
    outs[0][...] = emb
    for i in range(nderiv):
        outs[1 + i][...] = jnp.dot(
            emb, wts[2 * i][...], preferred_element_type=jnp.float32
        ) + wts[2 * i + 1][...]


def _tc_embed(x, lng, lnb, w1t, b1, w2t, b2, derivs, blk):
    n, f = x.shape
    grid = (n // blk,)
    full = lambda a: pl.BlockSpec(a.shape, lambda i: (0,) * a.ndim)
    args = [x, lng, lnb, w1t, b1, w2t, b2]
    in_specs = [
        pl.BlockSpec((blk, f), lambda i: (i, 0)),
        full(lng), full(lnb), full(w1t), full(b1), full(w2t), full(b2),
    ]
    body = functools.partial(_embed_body, len(derivs))
    for wt, bias in derivs:
        args.extend([wt, bias])
        in_specs.extend([full(wt), full(bias)])
    n_out = 1 + len(derivs)
    return pl.pallas_call(
        body,
        grid=grid,
        in_specs=in_specs,
        out_specs=[pl.BlockSpec((blk, EMB), lambda i: (i, 0))] * n_out,
        out_shape=[jax.ShapeDtypeStruct((n, EMB), jnp.float32)] * n_out,
    )(*args)


def _edge_body(m_ref, g_ref, b_ref, wft_ref, bf_ref, lo_ref, hi_ref):
    t = jnp.maximum(_ln_block(m_ref[...], g_ref[...], b_ref[...]), 0.0)
    mf = jnp.dot(t, wft_ref[...], preferred_element_type=jnp.float32) + bf_ref[...]
    lo_ref[...] = mf[:, : EMB // 2]
    hi_ref[...] = mf[:, EMB // 2 :]


def _tc_edge(m, g, b, wft, bf, blk):
    e = m.shape[0]
    grid = (e // blk,)
    full = lambda a: pl.BlockSpec(a.shape, lambda i: (0,) * a.ndim)
    return pl.pallas_call(
        _edge_body,
        grid=grid,
        in_specs=[
            pl.BlockSpec((blk, EMB), lambda i: (i, 0)),
            full(g), full(b), full(wft), full(bf),
        ],
        out_specs=[
            pl.BlockSpec((blk, EMB // 2), lambda i: (i, 0)),
            pl.BlockSpec((blk, EMB // 2), lambda i: (i, 0)),
        ],
        out_shape=[
            jax.ShapeDtypeStruct((e, EMB // 2), jnp.float32),
            jax.ShapeDtypeStruct((e, EMB // 2), jnp.float32),
        ],
    )(m, g, b, wft, bf)


def _post_node(alo_ref, ahi_ref, r_ref, png_ref, pnb_ref, wo1t_ref, bo1_ref,
               wo2t_ref, bo2_ref, lng_ref, lnb_ref):
    agg = jnp.concatenate([alo_ref[...], ahi_ref[...]], axis=-1)
    a = _ln_block(agg, png_ref[...], pnb_ref[...])
    h = jnp.concatenate([a, r_ref[...]], axis=-1)
    h = jnp.maximum(
        jnp.dot(h, wo1t_ref[...], preferred_element_type=jnp.float32) + bo1_ref[...], 0.0)
    x = jnp.dot(h, wo2t_ref[...], preferred_element_type=jnp.float32) + bo2_ref[...]
    return _ln_block(x, lng_ref[...], lnb_ref[...])


def _post_body(nderiv, *refs):
    core, rest = refs[:11], refs[11:]
    x = _post_node(*core)
    wts = rest[: 2 * nderiv]
    outs = rest[2 * nderiv :]
    outs[0][...] = x
    for i in range(nderiv):
        outs[1 + i][...] = jnp.dot(
            x, wts[2 * i][...], preferred_element_type=jnp.float32
        ) + wts[2 * i + 1][...]


def _tc_post(alo, ahi, r_emb, png, pnb, wo1t, bo1, wo2t, bo2, lng, lnb,
             derivs, blk):
    n = r_emb.shape[0]
    grid = (n // blk,)
    full = lambda a: pl.BlockSpec(a.shape, lambda i: (0,) * a.ndim)
    args = [alo, ahi, r_emb, png, pnb, wo1t, bo1, wo2t, bo2, lng, lnb]
    in_specs = [
        pl.BlockSpec((blk, EMB // 2), lambda i: (i, 0)),
        pl.BlockSpec((blk, EMB // 2), lambda i: (i, 0)),
        pl.BlockSpec((blk, EMB), lambda i: (i, 0)),
        full(png), full(pnb), full(wo1t), full(bo1),
        full(wo2t), full(bo2), full(lng), full(lnb),
    ]
    for wt, bias in derivs:
        args.extend([wt, bias])
        in_specs.extend([full(wt), full(bias)])
    n_out = 1 + len(derivs)
    return pl.pallas_call(
        functools.partial(_post_body, len(derivs)),
        grid=grid,
        in_specs=in_specs,
        out_specs=[pl.BlockSpec((blk, EMB), lambda i: (i, 0))] * n_out,
        out_shape=[jax.ShapeDtypeStruct((n, EMB), jnp.float32)] * n_out,
    )(*args)


def _post_final_body(alo_ref, ahi_ref, r_ref, png_ref, pnb_ref, wo1t_ref,
                     bo1_ref, wo2t_ref, bo2_ref, lng_ref, lnb_ref,
                     w1t_ref, b1_ref, w2t_ref, o_ref):
    x = _post_node(alo_ref, ahi_ref, r_ref, png_ref, pnb_ref, wo1t_ref,
                   bo1_ref, wo2t_ref, bo2_ref, lng_ref, lnb_ref)
    h = jnp.maximum(
        jnp.dot(x, w1t_ref[...], preferred_element_type=jnp.float32)
        + b1_ref[...], 0.0)
    o_ref[...] = jnp.dot(h, w2t_ref[...], preferred_element_type=jnp.float32)


def _tc_post_final(alo, ahi, r_emb, png, pnb, wo1t, bo1, wo2t, bo2, lng, lnb,
                   w1t, b1, w2t, blk):
    n = r_emb.shape[0]
    grid = (n // blk,)
    full = lambda a: pl.BlockSpec(a.shape, lambda i: (0,) * a.ndim)
    return pl.pallas_call(
        _post_final_body,
        grid=grid,
        in_specs=[
            pl.BlockSpec((blk, EMB // 2), lambda i: (i, 0)),
            pl.BlockSpec((blk, EMB // 2), lambda i: (i, 0)),
            pl.BlockSpec((blk, EMB), lambda i: (i, 0)),
            full(png), full(pnb), full(wo1t), full(bo1),
            full(wo2t), full(bo2), full(lng), full(lnb),
            full(w1t), full(b1), full(w2t),
        ],
        out_specs=pl.BlockSpec((blk, 1), lambda i: (i, 0)),
        out_shape=jax.ShapeDtypeStruct((n, 1), jnp.float32),
    )(alo, ahi, r_emb, png, pnb, wo1t, bo1, wo2t, bo2, lng, lnb, w1t, b1, w2t)


@functools.lru_cache(maxsize=None)
def _make_sc_gather(n_a, n_b, e_pad):
    info = plsc.get_sparse_core_info()
    ncores, nsub = info.num_cores, info.num_subcores
    nw = ncores * nsub
    per_w = e_pad // nw
    C = IDXW
    n_chunks = per_w // C
    NB = 4
    n_outer = n_chunks // NB
    mesh = plsc.VectorSubcoreMesh(core_axis_name="c", subcore_axis_name="s")

    @functools.partial(
        pl.kernel,
        mesh=mesh,
        compiler_params=pltpu.CompilerParams(use_tc_tiling_on_sc=False),
        out_type=jax.ShapeDtypeStruct((e_pad, EMB), jnp.float32),
        scratch_types=[
            pltpu.VMEM((n_chunks, IDXW), jnp.int32),
            pltpu.VMEM((n_chunks, IDXW), jnp.int32),
        ]
        + [pltpu.VMEM((C, EMB), jnp.float32)] * (2 * NB)
        + [pltpu.SemaphoreType.DMA] * (2 * NB),
    )
    def k(a_hbm, b_hbm, dst_hbm, src_hbm, m_hbm, dsti, srci, *bs):
        ras = bs[:NB]
        rbs = bs[NB : 2 * NB]
        gss = bs[2 * NB : 3 * NB]
        wss = bs[3 * NB :]
        wid = lax.axis_index("s") * ncores + lax.axis_index("c")
        base0 = wid * per_w
        irow0 = pl.multiple_of(wid * n_chunks, 8)
        pltpu.sync_copy(dst_hbm.at[pl.ds(irow0, n_chunks)], dsti)
        pltpu.sync_copy(src_hbm.at[pl.ds(irow0, n_chunks)], srci)

        def issue_gather(j, b):
            pltpu.async_copy(a_hbm.at[dsti.at[j]], ras[b], gss[b])
            pltpu.async_copy(b_hbm.at[srci.at[j]], rbs[b], gss[b])

        def wait_gather(j, b):
            pltpu.make_async_copy(a_hbm.at[dsti.at[j]], ras[b], gss[b]).wait()
            pltpu.make_async_copy(b_hbm.at[srci.at[j]], rbs[b], gss[b]).wait()

        def issue_wb(j, b):
            base = pl.multiple_of(base0 + j * C, C)
            pltpu.async_copy(ras[b], m_hbm.at[pl.ds(base, C)], wss[b])

        def wait_wb(j, b):
            base = pl.multiple_of(base0 + j * C, C)
            pltpu.make_async_copy(ras[b], m_hbm.at[pl.ds(base, C)], wss[b]).wait()

        issue_gather(0, 0)
        issue_gather(1, 1)

        def outer(jj, carry):
            for b in range(NB):
                j = jj * NB + b
                bn = (b + 2) % NB

                @pl.when(j >= 2)
                def _():
                    wait_wb(j - 2, bn)

                @pl.when(j + 2 < n_chunks)
                def _():
                    issue_gather(j + 2, bn)

                wait_gather(j, b)

                def add_row(i, c):
                    for c4 in range(EMB // LANES):
                        sl = pl.ds(c4 * LANES, LANES)
                        ras[b][i, sl] = ras[b][i, sl] + rbs[b][i, sl]
                    return c

                lax.fori_loop(0, C, add_row, 0)
                issue_wb(j, b)
            return carry

        lax.fori_loop(0, n_outer, outer, 0)
        wait_wb(n_chunks - 2, (n_chunks - 2) % NB)
        wait_wb(n_chunks - 1, (n_chunks - 1) % NB)

    return k


@functools.lru_cache(maxsize=None)
def _make_sc_scatter(n_nodes, e_pad):
    info = plsc.get_sparse_core_info()
    ncores, nsub = info.num_cores, info.num_subcores
    half = EMB // 2
    per_tile_e = e_pad // nsub
    C = 256
    n_chunks = per_tile_e // C
    nidx = C // IDXW
    unit = 200
    n_units = n_nodes // unit
    units_per_tile = (n_units + nsub - 1) // nsub
    mesh = plsc.VectorSubcoreMesh(core_axis_name="c", subcore_axis_name="s")

    out_sd = jax.ShapeDtypeStruct((n_nodes, half), jnp.float32)

    @functools.partial(
        pl.kernel,
        mesh=mesh,
        compiler_params=pltpu.CompilerParams(use_tc_tiling_on_sc=False),
        out_type=(out_sd, out_sd),
        scratch_types=[
            pltpu.VMEM((nidx, IDXW), jnp.int32),
            pltpu.VMEM((nidx, IDXW), jnp.int32),
            pltpu.VMEM((C, half), jnp.float32),
            pltpu.VMEM((C, half), jnp.float32),
            pltpu.VMEM((unit, half), jnp.float32),
            pltpu.MemorySpace.VMEM_SHARED((n_nodes + TRASH, half), jnp.float32),
            pltpu.SemaphoreType.DMA,
            pltpu.SemaphoreType.DMA,
            pltpu.SemaphoreType.DMA,
            pltpu.SemaphoreType.DMA,
        ],
    )
    def k(lo_hbm, hi_hbm, dst_hbm, out_lo, out_hi, idx0, idx1, rows0, rows1,
          zbuf, agg_sh, rs0, rs1, ss0, ss1):
        ibufs = (idx0, idx1)
        rbufs = (rows0, rows1)
        rsem = (rs0, rs1)
        ssem = (ss0, ss1)
        core = lax.axis_index("c")
        sub = lax.axis_index("s")

        def zb(i, c):
            for c2 in range(half // LANES):
                zbuf[i, pl.ds(c2 * LANES, LANES)] = jnp.zeros((LANES,), jnp.float32)
            return c

        lax.fori_loop(0, unit, zb, 0)

        def zz(i, c):
            u = i * nsub + sub

            @pl.when(u < n_units)
            def _():
                off = pl.multiple_of(u * unit, 8)
                pltpu.sync_copy(zbuf, agg_sh.at[pl.ds(off, unit)])

            return c

        lax.fori_loop(0, units_per_tile, zz, 0)

        @pl.when(sub == 0)
        def _():
            pltpu.sync_copy(zbuf.at[pl.ds(0, TRASH)],
                            agg_sh.at[pl.ds(n_nodes, TRASH)])

        plsc.subcore_barrier()

        base0 = sub * per_tile_e
        irow_base = sub * (per_tile_e // IDXW)

        def issue_read(j, b):
            base = pl.multiple_of(base0 + j * C, C)
            irow = pl.multiple_of(irow_base + j * nidx, nidx)
            pltpu.async_copy(dst_hbm.at[pl.ds(irow, nidx)], ibufs[b], rsem[b])

            @pl.when(core == 0)
            def _():
                pltpu.async_copy(lo_hbm.at[pl.ds(base, C)], rbufs[b], rsem[b])

            @pl.when(core == 1)
            def _():
                pltpu.async_copy(hi_hbm.at[pl.ds(base, C)], rbufs[b], rsem[b])

        def wait_read(j, b):
            base = pl.multiple_of(base0 + j * C, C)
            irow = pl.multiple_of(irow_base + j * nidx, nidx)
            pltpu.make_async_copy(
                dst_hbm.at[pl.ds(irow, nidx)], ibufs[b], rsem[b]).wait()
            pltpu.make_async_copy(
                lo_hbm.at[pl.ds(base, C)], rbufs[b], rsem[b]).wait()

        def issue_scat(j, b):
            for q in range(nidx):
                pltpu.async_copy(
                    rbufs[b].at[pl.ds(q * IDXW, IDXW)],
                    agg_sh.at[ibufs[b].at[q]], ssem[b], add=True)

        def wait_scat(j, b):
            for q in range(nidx):
                pltpu.make_async_copy(
                    rbufs[b].at[pl.ds(q * IDXW, IDXW)],
                    agg_sh.at[ibufs[b].at[q]], ssem[b]).wait()

        issue_read(0, 0)

        def outer(jj, carry):
            for b in range(2):
                j = jj * 2 + b
                wait_read(j, b)
                issue_scat(j, b)

                @pl.when(j >= 1)
                def _():
                    wait_scat(j - 1, 1 - b)

                @pl.when(j + 1 < n_chunks)
                def _():
                    issue_read(j + 1, 1 - b)

            return carry

        lax.fori_loop(0, n_chunks // 2, outer, 0)
        wait_scat(n_chunks - 1, (n_chunks - 1) % 2)
        plsc.subcore_barrier()

        def wb(i, c):
            u = i * nsub + sub

            @pl.when(u < n_units)
            def _():
                sl = pl.ds(pl.multiple_of(u * unit, 8), unit)

                @pl.when(core == 0)
                def _():
                    pltpu.sync_copy(agg_sh.at[sl], out_lo.at[sl])

                @pl.when(core == 1)
                def _():
                    pltpu.sync_copy(agg_sh.at[sl], out_hi.at[sl])

            return c

        lax.fori_loop(0, units_per_tile, wb, 0)

    return k


def _edge_stage(a_nodes, b_nodes, dst_g, src_g, dst_s, e_pad, pc):
    n_r = a_nodes.shape[0]
    n_l = b_nodes.shape[0]
    m = _make_sc_gather(n_r, n_l, e_pad)(a_nodes, b_nodes, dst_g, src_g)
    mf_lo, mf_hi = _tc_edge(
        m, pc["ln_f_g"].reshape(1, EMB), pc["ln_f_b"].reshape(1, EMB),
        pc["Wf"].T, pc["bf"].reshape(1, EMB), 4096)
    return _make_sc_scatter(n_r, e_pad)(mf_lo, mf_hi, dst_s)


def _bias_a(pc, e_bias):
    return (pc["bl"] + e_bias * pc["We"][:, 0]).reshape(1, EMB)


def kernel(constraint_features, edge_indices, edge_features, variable_features, params):
    del edge_features
    p = params
    n_c = constraint_features.shape[0]
    n_v = variable_features.shape[0]
    e = edge_indices.shape[1]
    blk = 2000
    zero64 = jnp.zeros((1, EMB), jnp.float32)

    lane_chunk = 32 * GROUP
    e_pad = ((e + lane_chunk - 1) // lane_chunk) * lane_chunk

    ei0 = edge_indices[0].astype(jnp.int32)
    ei1 = edge_indices[1].astype(jnp.int32)
    pad = e_pad - e

    def pad_to(v, fill):
        return jnp.pad(v, (0, pad), constant_values=fill).reshape(e_pad // IDXW, IDXW)

    ei0_g = pad_to(ei0, 0)
    ei1_g = pad_to(ei1, 0)
    ei0_s = pad_to(ei0, n_c)
    ei1_s = pad_to(ei1, n_v)

    e_bias = p["ln_e_b"][0]
    pvc = p["conv_vc"]
    pcv = p["conv_cv"]

    cemb, a_vc = _tc_embed(
        constraint_features,
        p["ln_c_in_g"].reshape(1, -1), p["ln_c_in_b"].reshape(1, -1),
        p["Wc1"].T, p["bc1"].reshape(1, EMB), p["Wc2"].T, p["bc2"].reshape(1, EMB),
        [(pvc["Wl"].T, _bias_a(pvc, e_bias))], blk)
    vemb, b_vc, a_cv = _tc_embed(
        variable_features,
        p["ln_v_in_g"].reshape(1, -1), p["ln_v_in_b"].reshape(1, -1),
        p["Wv1"].T, p["bv1"].reshape(1, EMB), p["Wv2"].T, p["bv2"].reshape(1, EMB),
        [(pvc["Wr"].T, zero64), (pcv["Wl"].T, _bias_a(pcv, e_bias))], blk)

    agg_lo, agg_hi = _edge_stage(a_vc, b_vc, ei0_g, ei1_g, ei0_s, e_pad, pvc)
    c2, b_cv = _tc_post(
        agg_lo, agg_hi, cemb,
        pvc["ln_post_g"].reshape(1, EMB), pvc["ln_post_b"].reshape(1, EMB),
        pvc["Wo1"].T, pvc["bo1"].reshape(1, EMB),
        pvc["Wo2"].T, pvc["bo2"].reshape(1, EMB),
        p["ln_c_g"].reshape(1, EMB), p["ln_c_b"].reshape(1, EMB),
        [(pcv["Wr"].T, zero64)], blk)

    agg_lo2, agg_hi2 = _edge_stage(a_cv, b_cv, ei1_g, ei0_g, ei1_s, e_pad, pcv)
    out = _tc_post_final(
        agg_lo2, agg_hi2, vemb,
        pcv["ln_post_g"].reshape(1, EMB), pcv["ln_post_b"].reshape(1, EMB),
        pcv["Wo1"].T, pcv["bo1"].reshape(1, EMB),
        pcv["Wo2"].T, pcv["bo2"].reshape(1, EMB),
        p["ln_v_g"].reshape(1, EMB), p["ln_v_b"].reshape(1, EMB),
        p["Wout1"].T, p["bout1"].reshape(1, EMB), p["Wout2"].T, blk)
    return out[:, 0]


# speedup vs baseline: 2.1511x; 1.0011x over previous
import functools

import jax
import jax.numpy as jnp
from jax import lax
from jax.experimental import pallas as pl
from jax.experimental.pallas import tpu as pltpu
from jax.experimental.pallas import tpu_sc as plsc

EMB = 64
LANES = 16
IDXW = 128
CHUNK = 512
GROUP = 1024
TRASH = 128

_EPS = 1e-5


def _ln_block(x, g, b):
    mu = jnp.mean(x, axis=-1, keepdims=True)
    var = jnp.mean((x - mu) * (x - mu), axis=-1, keepdims=True)
    return (x - mu) * lax.rsqrt(var + _EPS) * g + b


def _embed_body(nderiv, x_ref, lng_ref, lnb_ref, w1_ref, b1_ref, w2_ref, b2_ref,
                *rest):
    x = _ln_block(x_ref[...], lng_ref[...], lnb_ref[...])
    h = jnp.maximum(
        jnp.dot(x, w1_ref[...], preferred_element_type=jnp.float32) + b1_ref[...], 0.0)
    emb = jnp.maximum(
        jnp.dot(h, w2_ref[...], preferred_element_type=jnp.float32) + b2_ref[...], 0.0)
    wts = rest[: 2 * nderiv]
    outs = rest[2 * nderiv :]
    outs[0][...] = emb
    for i in range(nderiv):
        outs[1 + i][...] = jnp.dot(
            emb, wts[2 * i][...], preferred_element_type=jnp.float32
        ) + wts[2 * i + 1][...]


def _tc_embed(x, lng, lnb, w1t, b1, w2t, b2, derivs, blk):
    n, f = x.shape
    grid = (n // blk,)
    full = lambda a: pl.BlockSpec(a.shape, lambda i: (0,) * a.ndim)
    args = [x, lng, lnb, w1t, b1, w2t, b2]
    in_specs = [
        pl.BlockSpec((blk, f), lambda i: (i, 0)),
        full(lng), full(lnb), full(w1t), full(b1), full(w2t), full(b2),
    ]
    body = functools.partial(_embed_body, len(derivs))
    for wt, bias in derivs:
        args.extend([wt, bias])
        in_specs.extend([full(wt), full(bias)])
    n_out = 1 + len(derivs)
    return pl.pallas_call(
        body,
        grid=grid,
        in_specs=in_specs,
        out_specs=[pl.BlockSpec((blk, EMB), lambda i: (i, 0))] * n_out,
        out_shape=[jax.ShapeDtypeStruct((n, EMB), jnp.float32)] * n_out,
    )(*args)


def _edge_body(m_ref, g_ref, b_ref, wft_ref, bf_ref, lo_ref, hi_ref):
    t = jnp.maximum(_ln_block(m_ref[...], g_ref[...], b_ref[...]), 0.0)
    mf = jnp.dot(t, wft_ref[...], preferred_element_type=jnp.float32) + bf_ref[...]
    lo_ref[...] = mf[:, : EMB // 2]
    hi_ref[...] = mf[:, EMB // 2 :]


def _tc_edge(m, g, b, wft, bf, blk):
    e = m.shape[0]
    grid = (e // blk,)
    full = lambda a: pl.BlockSpec(a.shape, lambda i: (0,) * a.ndim)
    return pl.pallas_call(
        _edge_body,
        grid=grid,
        in_specs=[
            pl.BlockSpec((blk, EMB), lambda i: (i, 0)),
            full(g), full(b), full(wft), full(bf),
        ],
        out_specs=[
            pl.BlockSpec((blk, EMB // 2), lambda i: (i, 0)),
            pl.BlockSpec((blk, EMB // 2), lambda i: (i, 0)),
        ],
        out_shape=[
            jax.ShapeDtypeStruct((e, EMB // 2), jnp.float32),
            jax.ShapeDtypeStruct((e, EMB // 2), jnp.float32),
        ],
    )(m, g, b, wft, bf)


def _post_node(alo_ref, ahi_ref, r_ref, png_ref, pnb_ref, wo1t_ref, bo1_ref,
               wo2t_ref, bo2_ref, lng_ref, lnb_ref):
    agg = jnp.concatenate([alo_ref[...], ahi_ref[...]], axis=-1)
    a = _ln_block(agg, png_ref[...], pnb_ref[...])
    h = jnp.concatenate([a, r_ref[...]], axis=-1)
    h = jnp.maximum(
        jnp.dot(h, wo1t_ref[...], preferred_element_type=jnp.float32) + bo1_ref[...], 0.0)
    x = jnp.dot(h, wo2t_ref[...], preferred_element_type=jnp.float32) + bo2_ref[...]
    return _ln_block(x, lng_ref[...], lnb_ref[...])


def _post_body(nderiv, *refs):
    core, rest = refs[:11], refs[11:]
    x = _post_node(*core)
    wts = rest[: 2 * nderiv]
    outs = rest[2 * nderiv :]
    outs[0][...] = x
    for i in range(nderiv):
        outs[1 + i][...] = jnp.dot(
            x, wts[2 * i][...], preferred_element_type=jnp.float32
        ) + wts[2 * i + 1][...]


def _tc_post(alo, ahi, r_emb, png, pnb, wo1t, bo1, wo2t, bo2, lng, lnb,
             derivs, blk):
    n = r_emb.shape[0]
    grid = (n // blk,)
    full = lambda a: pl.BlockSpec(a.shape, lambda i: (0,) * a.ndim)
    args = [alo, ahi, r_emb, png, pnb, wo1t, bo1, wo2t, bo2, lng, lnb]
    in_specs = [
        pl.BlockSpec((blk, EMB // 2), lambda i: (i, 0)),
        pl.BlockSpec((blk, EMB // 2), lambda i: (i, 0)),
        pl.BlockSpec((blk, EMB), lambda i: (i, 0)),
        full(png), full(pnb), full(wo1t), full(bo1),
        full(wo2t), full(bo2), full(lng), full(lnb),
    ]
    for wt, bias in derivs:
        args.extend([wt, bias])
        in_specs.extend([full(wt), full(bias)])
    n_out = 1 + len(derivs)
    return pl.pallas_call(
        functools.partial(_post_body, len(derivs)),
        grid=grid,
        in_specs=in_specs,
        out_specs=[pl.BlockSpec((blk, EMB), lambda i: (i, 0))] * n_out,
        out_shape=[jax.ShapeDtypeStruct((n, EMB), jnp.float32)] * n_out,
    )(*args)


def _post_final_body(alo_ref, ahi_ref, r_ref, png_ref, pnb_ref, wo1t_ref,
                     bo1_ref, wo2t_ref, bo2_ref, lng_ref, lnb_ref,
                     w1t_ref, b1_ref, w2t_ref, o_ref):
    x = _post_node(alo_ref, ahi_ref, r_ref, png_ref, pnb_ref, wo1t_ref,
                   bo1_ref, wo2t_ref, bo2_ref, lng_ref, lnb_ref)
    h = jnp.maximum(
        jnp.dot(x, w1t_ref[...], preferred_element_type=jnp.float32)
        + b1_ref[...], 0.0)
    o_ref[...] = jnp.dot(h, w2t_ref[...], preferred_element_type=jnp.float32)


def _tc_post_final(alo, ahi, r_emb, png, pnb, wo1t, bo1, wo2t, bo2, lng, lnb,
                   w1t, b1, w2t, blk):
    n = r_emb.shape[0]
    grid = (n // blk,)
    full = lambda a: pl.BlockSpec(a.shape, lambda i: (0,) * a.ndim)
    return pl.pallas_call(
        _post_final_body,
        grid=grid,
        in_specs=[
            pl.BlockSpec((blk, EMB // 2), lambda i: (i, 0)),
            pl.BlockSpec((blk, EMB // 2), lambda i: (i, 0)),
            pl.BlockSpec((blk, EMB), lambda i: (i, 0)),
            full(png), full(pnb), full(wo1t), full(bo1),
            full(wo2t), full(bo2), full(lng), full(lnb),
            full(w1t), full(b1), full(w2t),
        ],
        out_specs=pl.BlockSpec((blk, 1), lambda i: (i, 0)),
        out_shape=jax.ShapeDtypeStruct((n, 1), jnp.float32),
    )(alo, ahi, r_emb, png, pnb, wo1t, bo1, wo2t, bo2, lng, lnb, w1t, b1, w2t)


@functools.lru_cache(maxsize=None)
def _make_sc_gather(n_a, n_b, e_pad):
    info = plsc.get_sparse_core_info()
    ncores, nsub = info.num_cores, info.num_subcores
    nw = ncores * nsub
    per_w = e_pad // nw
    C = IDXW
    n_chunks = per_w // C
    NB = 4
    n_outer = n_chunks // NB
    mesh = plsc.VectorSubcoreMesh(core_axis_name="c", subcore_axis_name="s")

    @functools.partial(
        pl.kernel,
        mesh=mesh,
        compiler_params=pltpu.CompilerParams(use_tc_tiling_on_sc=False),
        out_type=jax.ShapeDtypeStruct((e_pad, EMB), jnp.float32),
        scratch_types=[
            pltpu.VMEM((n_chunks, IDXW), jnp.int32),
            pltpu.VMEM((n_chunks, IDXW), jnp.int32),
        ]
        + [pltpu.VMEM((C, EMB), jnp.float32)] * (2 * NB)
        + [pltpu.SemaphoreType.DMA] * (2 * NB),
    )
    def k(a_hbm, b_hbm, dst_hbm, src_hbm, m_hbm, dsti, srci, *bs):
        ras = bs[:NB]
        rbs = bs[NB : 2 * NB]
        gss = bs[2 * NB : 3 * NB]
        wss = bs[3 * NB :]
        wid = lax.axis_index("s") * ncores + lax.axis_index("c")
        base0 = wid * per_w
        irow0 = pl.multiple_of(wid * n_chunks, 8)
        pltpu.sync_copy(dst_hbm.at[pl.ds(irow0, n_chunks)], dsti)
        pltpu.sync_copy(src_hbm.at[pl.ds(irow0, n_chunks)], srci)

        def issue_gather(j, b):
            pltpu.async_copy(a_hbm.at[dsti.at[j]], ras[b], gss[b])
            pltpu.async_copy(b_hbm.at[srci.at[j]], rbs[b], gss[b])

        def wait_gather(j, b):
            pltpu.make_async_copy(a_hbm.at[dsti.at[j]], ras[b], gss[b]).wait()
            pltpu.make_async_copy(b_hbm.at[srci.at[j]], rbs[b], gss[b]).wait()

        def issue_wb(j, b):
            base = pl.multiple_of(base0 + j * C, C)
            pltpu.async_copy(ras[b], m_hbm.at[pl.ds(base, C)], wss[b])

        def wait_wb(j, b):
            base = pl.multiple_of(base0 + j * C, C)
            pltpu.make_async_copy(ras[b], m_hbm.at[pl.ds(base, C)], wss[b]).wait()

        issue_gather(0, 0)
        issue_gather(1, 1)

        def outer(jj, carry):
            for b in range(NB):
                j = jj * NB + b
                bn = (b + 2) % NB

                @pl.when(j >= 2)
                def _():
                    wait_wb(j - 2, bn)

                @pl.when(j + 2 < n_chunks)
                def _():
                    issue_gather(j + 2, bn)

                wait_gather(j, b)
                unroll = 8

                def add_rows(i8, c):
                    for r in range(unroll):
                        i = i8 * unroll + r
                        for c4 in range(EMB // LANES):
                            sl = pl.ds(c4 * LANES, LANES)
                            ras[b][i, sl] = ras[b][i, sl] + rbs[b][i, sl]
                    return c

                lax.fori_loop(0, C // unroll, add_rows, 0)
                issue_wb(j, b)
            return carry

        lax.fori_loop(0, n_outer, outer, 0)
        wait_wb(n_chunks - 2, (n_chunks - 2) % NB)
        wait_wb(n_chunks - 1, (n_chunks - 1) % NB)

    return k


@functools.lru_cache(maxsize=None)
def _make_sc_scatter(n_nodes, e_pad):
    info = plsc.get_sparse_core_info()
    ncores, nsub = info.num_cores, info.num_subcores
    half = EMB // 2
    per_tile_e = e_pad // nsub
    C = 256
    n_chunks = per_tile_e // C
    nidx = C // IDXW
    unit = 200
    n_units = n_nodes // unit
    units_per_tile = (n_units + nsub - 1) // nsub
    mesh = plsc.VectorSubcoreMesh(core_axis_name="c", subcore_axis_name="s")

    out_sd = jax.ShapeDtypeStruct((n_nodes, half), jnp.float32)

    @functools.partial(
        pl.kernel,
        mesh=mesh,
        compiler_params=pltpu.CompilerParams(use_tc_tiling_on_sc=False),
        out_type=(out_sd, out_sd),
        scratch_types=[
            pltpu.VMEM((nidx, IDXW), jnp.int32),
            pltpu.VMEM((nidx, IDXW), jnp.int32),
            pltpu.VMEM((C, half), jnp.float32),
            pltpu.VMEM((C, half), jnp.float32),
            pltpu.VMEM((unit, half), jnp.float32),
            pltpu.MemorySpace.VMEM_SHARED((n_nodes + TRASH, half), jnp.float32),
            pltpu.SemaphoreType.DMA,
            pltpu.SemaphoreType.DMA,
            pltpu.SemaphoreType.DMA,
            pltpu.SemaphoreType.DMA,
        ],
    )
    def k(lo_hbm, hi_hbm, dst_hbm, out_lo, out_hi, idx0, idx1, rows0, rows1,
          zbuf, agg_sh, rs0, rs1, ss0, ss1):
        ibufs = (idx0, idx1)
        rbufs = (rows0, rows1)
        rsem = (rs0, rs1)
        ssem = (ss0, ss1)
        core = lax.axis_index("c")
        sub = lax.axis_index("s")

        def zb(i, c):
            for c2 in range(half // LANES):
                zbuf[i, pl.ds(c2 * LANES, LANES)] = jnp.zeros((LANES,), jnp.float32)
            return c

        lax.fori_loop(0, unit, zb, 0)

        def zz(i, c):
            u = i * nsub + sub

            @pl.when(u < n_units)
            def _():
                off = pl.multiple_of(u * unit, 8)
                pltpu.sync_copy(zbuf, agg_sh.at[pl.ds(off, unit)])

            return c

        lax.fori_loop(0, units_per_tile, zz, 0)

        @pl.when(sub == 0)
        def _():
            pltpu.sync_copy(zbuf.at[pl.ds(0, TRASH)],
                            agg_sh.at[pl.ds(n_nodes, TRASH)])

        plsc.subcore_barrier()

        base0 = sub * per_tile_e
        irow_base = sub * (per_tile_e // IDXW)

        def issue_read(j, b):
            base = pl.multiple_of(base0 + j * C, C)
            irow = pl.multiple_of(irow_base + j * nidx, nidx)
            pltpu.async_copy(dst_hbm.at[pl.ds(irow, nidx)], ibufs[b], rsem[b])

            @pl.when(core == 0)
            def _():
                pltpu.async_copy(lo_hbm.at[pl.ds(base, C)], rbufs[b], rsem[b])

            @pl.when(core == 1)
            def _():
                pltpu.async_copy(hi_hbm.at[pl.ds(base, C)], rbufs[b], rsem[b])

        def wait_read(j, b):
            base = pl.multiple_of(base0 + j * C, C)
            irow = pl.multiple_of(irow_base + j * nidx, nidx)
            pltpu.make_async_copy(
                dst_hbm.at[pl.ds(irow, nidx)], ibufs[b], rsem[b]).wait()
            pltpu.make_async_copy(
                lo_hbm.at[pl.ds(base, C)], rbufs[b], rsem[b]).wait()

        def issue_scat(j, b):
            for q in range(nidx):
                pltpu.async_copy(
                    rbufs[b].at[pl.ds(q * IDXW, IDXW)],
                    agg_sh.at[ibufs[b].at[q]], ssem[b], add=True)

        def wait_scat(j, b):
            for q in range(nidx):
                pltpu.make_async_copy(
                    rbufs[b].at[pl.ds(q * IDXW, IDXW)],
                    agg_sh.at[ibufs[b].at[q]], ssem[b]).wait()

        issue_read(0, 0)

        def outer(jj, carry):
            for b in range(2):
                j = jj * 2 + b
                wait_read(j, b)
                issue_scat(j, b)

                @pl.when(j >= 1)
                def _():
                    wait_scat(j - 1, 1 - b)

                @pl.when(j + 1 < n_chunks)
                def _():
                    issue_read(j + 1, 1 - b)

            return carry

        lax.fori_loop(0, n_chunks // 2, outer, 0)
        wait_scat(n_chunks - 1, (n_chunks - 1) % 2)
        plsc.subcore_barrier()

        def wb(i, c):
            u = i * nsub + sub

            @pl.when(u < n_units)
            def _():
                sl = pl.ds(pl.multiple_of(u * unit, 8), unit)

                @pl.when(core == 0)
                def _():
                    pltpu.sync_copy(agg_sh.at[sl], out_lo.at[sl])

                @pl.when(core == 1)
                def _():
                    pltpu.sync_copy(agg_sh.at[sl], out_hi.at[sl])

            return c

        lax.fori_loop(0, units_per_tile, wb, 0)

    return k


def _edge_stage(a_nodes, b_nodes, dst_g, src_g, dst_s, e_pad, pc):
    n_r = a_nodes.shape[0]
    n_l = b_nodes.shape[0]
    m = _make_sc_gather(n_r, n_l, e_pad)(a_nodes, b_nodes, dst_g, src_g)
    mf_lo, mf_hi = _tc_edge(
        m, pc["ln_f_g"].reshape(1, EMB), pc["ln_f_b"].reshape(1, EMB),
        pc["Wf"].T, pc["bf"].reshape(1, EMB), 4096)
    return _make_sc_scatter(n_r, e_pad)(mf_lo, mf_hi, dst_s)


def _bias_a(pc, e_bias):
    return (pc["bl"] + e_bias * pc["We"][:, 0]).reshape(1, EMB)


def kernel(constraint_features, edge_indices, edge_features, variable_features, params):
    del edge_features
    p = params
    n_c = constraint_features.shape[0]
    n_v = variable_features.shape[0]
    e = edge_indices.shape[1]
    blk = 2000
    zero64 = jnp.zeros((1, EMB), jnp.float32)

    lane_chunk = 32 * GROUP
    e_pad = ((e + lane_chunk - 1) // lane_chunk) * lane_chunk

    ei0 = edge_indices[0].astype(jnp.int32)
    ei1 = edge_indices[1].astype(jnp.int32)
    pad = e_pad - e

    def pad_to(v, fill):
        return jnp.pad(v, (0, pad), constant_values=fill).reshape(e_pad // IDXW, IDXW)

    ei0_g = pad_to(ei0, 0)
    ei1_g = pad_to(ei1, 0)
    ei0_s = pad_to(ei0, n_c)
    ei1_s = pad_to(ei1, n_v)

    e_bias = p["ln_e_b"][0]
    pvc = p["conv_vc"]
    pcv = p["conv_cv"]

    cemb, a_vc = _tc_embed(
        constraint_features,
        p["ln_c_in_g"].reshape(1, -1), p["ln_c_in_b"].reshape(1, -1),
        p["Wc1"].T, p["bc1"].reshape(1, EMB), p["Wc2"].T, p["bc2"].reshape(1, EMB),
        [(pvc["Wl"].T, _bias_a(pvc, e_bias))], blk)
    vemb, b_vc, a_cv = _tc_embed(
        variable_features,
        p["ln_v_in_g"].reshape(1, -1), p["ln_v_in_b"].reshape(1, -1),
        p["Wv1"].T, p["bv1"].reshape(1, EMB), p["Wv2"].T, p["bv2"].reshape(1, EMB),
        [(pvc["Wr"].T, zero64), (pcv["Wl"].T, _bias_a(pcv, e_bias))], blk)

    agg_lo, agg_hi = _edge_stage(a_vc, b_vc, ei0_g, ei1_g, ei0_s, e_pad, pvc)
    c2, b_cv = _tc_post(
        agg_lo, agg_hi, cemb,
        pvc["ln_post_g"].reshape(1, EMB), pvc["ln_post_b"].reshape(1, EMB),
        pvc["Wo1"].T, pvc["bo1"].reshape(1, EMB),
        pvc["Wo2"].T, pvc["bo2"].reshape(1, EMB),
        p["ln_c_g"].reshape(1, EMB), p["ln_c_b"].reshape(1, EMB),
        [(pcv["Wr"].T, zero64)], blk)

    agg_lo2, agg_hi2 = _edge_stage(a_cv, b_cv, ei1_g, ei0_g, ei1_s, e_pad, pcv)
    out = _tc_post_final(
        agg_lo2, agg_hi2, vemb,
        pcv["ln_post_g"].reshape(1, EMB), pcv["ln_post_b"].reshape(1, EMB),
        pcv["Wo1"].T, pcv["bo1"].reshape(1, EMB),
        pcv["Wo2"].T, pcv["bo2"].reshape(1, EMB),
        p["ln_v_g"].reshape(1, EMB), p["ln_v_b"].reshape(1, EMB),
        p["Wout1"].T, p["bout1"].reshape(1, EMB), p["Wout2"].T, blk)
    return out[:, 0]


# speedup vs baseline: 2.3134x; 1.0755x over previous
import functools

import jax
import jax.numpy as jnp
from jax import lax
from jax.experimental import pallas as pl
from jax.experimental.pallas import tpu as pltpu
from jax.experimental.pallas import tpu_sc as plsc

EMB = 64
LANES = 16
IDXW = 128
CHUNK = 512
GROUP = 1024
TRASH = 128

_EPS = 1e-5


def _ln_block(x, g, b):
    mu = jnp.mean(x, axis=-1, keepdims=True)
    var = jnp.mean((x - mu) * (x - mu), axis=-1, keepdims=True)
    return (x - mu) * lax.rsqrt(var + _EPS) * g + b


def _embed_body(nderiv, x_ref, lng_ref, lnb_ref, w1_ref, b1_ref, w2_ref, b2_ref,
                *rest):
    x = _ln_block(x_ref[...], lng_ref[...], lnb_ref[...])
    h = jnp.maximum(
        jnp.dot(x, w1_ref[...], preferred_element_type=jnp.float32) + b1_ref[...], 0.0)
    emb = jnp.maximum(
        jnp.dot(h, w2_ref[...], preferred_element_type=jnp.float32) + b2_ref[...], 0.0)
    wts = rest[: 2 * nderiv]
    outs = rest[2 * nderiv :]
    outs[0][...] = emb
    for i in range(nderiv):
        outs[1 + i][...] = (jnp.dot(
            emb, wts[2 * i][...], preferred_element_type=jnp.float32
        ) + wts[2 * i + 1][...]).astype(jnp.bfloat16)


def _tc_embed(x, lng, lnb, w1t, b1, w2t, b2, derivs, blk):
    n, f = x.shape
    grid = (n // blk,)
    full = lambda a: pl.BlockSpec(a.shape, lambda i: (0,) * a.ndim)
    args = [x, lng, lnb, w1t, b1, w2t, b2]
    in_specs = [
        pl.BlockSpec((blk, f), lambda i: (i, 0)),
        full(lng), full(lnb), full(w1t), full(b1), full(w2t), full(b2),
    ]
    body = functools.partial(_embed_body, len(derivs))
    for wt, bias in derivs:
        args.extend([wt, bias])
        in_specs.extend([full(wt), full(bias)])
    n_out = 1 + len(derivs)
    return pl.pallas_call(
        body,
        grid=grid,
        in_specs=in_specs,
        out_specs=[pl.BlockSpec((blk, EMB), lambda i: (i, 0))] * n_out,
        out_shape=[jax.ShapeDtypeStruct((n, EMB), jnp.float32)]
        + [jax.ShapeDtypeStruct((n, EMB), jnp.bfloat16)] * len(derivs),
    )(*args)


def _edge_body(m_ref, g_ref, b_ref, wft_ref, bf_ref, lo_ref, hi_ref):
    m = m_ref[...].astype(jnp.float32)
    t = jnp.maximum(_ln_block(m, g_ref[...], b_ref[...]), 0.0)
    mf = jnp.dot(t, wft_ref[...], preferred_element_type=jnp.float32) + bf_ref[...]
    lo_ref[...] = mf[:, : EMB // 2]
    hi_ref[...] = mf[:, EMB // 2 :]


def _tc_edge(m, g, b, wft, bf, blk):
    e = m.shape[0]
    grid = (e // blk,)
    full = lambda a: pl.BlockSpec(a.shape, lambda i: (0,) * a.ndim)
    return pl.pallas_call(
        _edge_body,
        grid=grid,
        in_specs=[
            pl.BlockSpec((blk, EMB), lambda i: (i, 0)),
            full(g), full(b), full(wft), full(bf),
        ],
        out_specs=[
            pl.BlockSpec((blk, EMB // 2), lambda i: (i, 0)),
            pl.BlockSpec((blk, EMB // 2), lambda i: (i, 0)),
        ],
        out_shape=[
            jax.ShapeDtypeStruct((e, EMB // 2), jnp.float32),
            jax.ShapeDtypeStruct((e, EMB // 2), jnp.float32),
        ],
    )(m, g, b, wft, bf)


def _post_node(alo_ref, ahi_ref, r_ref, png_ref, pnb_ref, wo1t_ref, bo1_ref,
               wo2t_ref, bo2_ref, lng_ref, lnb_ref):
    agg = jnp.concatenate([alo_ref[...], ahi_ref[...]], axis=-1)
    a = _ln_block(agg, png_ref[...], pnb_ref[...])
    h = jnp.concatenate([a, r_ref[...]], axis=-1)
    h = jnp.maximum(
        jnp.dot(h, wo1t_ref[...], preferred_element_type=jnp.float32) + bo1_ref[...], 0.0)
    x = jnp.dot(h, wo2t_ref[...], preferred_element_type=jnp.float32) + bo2_ref[...]
    return _ln_block(x, lng_ref[...], lnb_ref[...])


def _post_body(nderiv, *refs):
    core, rest = refs[:11], refs[11:]
    x = _post_node(*core)
    wts = rest[: 2 * nderiv]
    outs = rest[2 * nderiv :]
    outs[0][...] = x
    for i in range(nderiv):
        outs[1 + i][...] = (jnp.dot(
            x, wts[2 * i][...], preferred_element_type=jnp.float32
        ) + wts[2 * i + 1][...]).astype(jnp.bfloat16)


def _tc_post(alo, ahi, r_emb, png, pnb, wo1t, bo1, wo2t, bo2, lng, lnb,
             derivs, blk):
    n = r_emb.shape[0]
    grid = (n // blk,)
    full = lambda a: pl.BlockSpec(a.shape, lambda i: (0,) * a.ndim)
    args = [alo, ahi, r_emb, png, pnb, wo1t, bo1, wo2t, bo2, lng, lnb]
    in_specs = [
        pl.BlockSpec((blk, EMB // 2), lambda i: (i, 0)),
        pl.BlockSpec((blk, EMB // 2), lambda i: (i, 0)),
        pl.BlockSpec((blk, EMB), lambda i: (i, 0)),
        full(png), full(pnb), full(wo1t), full(bo1),
        full(wo2t), full(bo2), full(lng), full(lnb),
    ]
    for wt, bias in derivs:
        args.extend([wt, bias])
        in_specs.extend([full(wt), full(bias)])
    n_out = 1 + len(derivs)
    return pl.pallas_call(
        functools.partial(_post_body, len(derivs)),
        grid=grid,
        in_specs=in_specs,
        out_specs=[pl.BlockSpec((blk, EMB), lambda i: (i, 0))] * n_out,
        out_shape=[jax.ShapeDtypeStruct((n, EMB), jnp.float32)]
        + [jax.ShapeDtypeStruct((n, EMB), jnp.bfloat16)] * len(derivs),
    )(*args)


def _post_final_body(alo_ref, ahi_ref, r_ref, png_ref, pnb_ref, wo1t_ref,
                     bo1_ref, wo2t_ref, bo2_ref, lng_ref, lnb_ref,
                     w1t_ref, b1_ref, w2t_ref, o_ref):
    x = _post_node(alo_ref, ahi_ref, r_ref, png_ref, pnb_ref, wo1t_ref,
                   bo1_ref, wo2t_ref, bo2_ref, lng_ref, lnb_ref)
    h = jnp.maximum(
        jnp.dot(x, w1t_ref[...], preferred_element_type=jnp.float32)
        + b1_ref[...], 0.0)
    o_ref[...] = jnp.dot(h, w2t_ref[...], preferred_element_type=jnp.float32)


def _tc_post_final(alo, ahi, r_emb, png, pnb, wo1t, bo1, wo2t, bo2, lng, lnb,
                   w1t, b1, w2t, blk):
    n = r_emb.shape[0]
    grid = (n // blk,)
    full = lambda a: pl.BlockSpec(a.shape, lambda i: (0,) * a.ndim)
    return pl.pallas_call(
        _post_final_body,
        grid=grid,
        in_specs=[
            pl.BlockSpec((blk, EMB // 2), lambda i: (i, 0)),
            pl.BlockSpec((blk, EMB // 2), lambda i: (i, 0)),
            pl.BlockSpec((blk, EMB), lambda i: (i, 0)),
            full(png), full(pnb), full(wo1t), full(bo1),
            full(wo2t), full(bo2), full(lng), full(lnb),
            full(w1t), full(b1), full(w2t),
        ],
        out_specs=pl.BlockSpec((blk, 1), lambda i: (i, 0)),
        out_shape=jax.ShapeDtypeStruct((n, 1), jnp.float32),
    )(alo, ahi, r_emb, png, pnb, wo1t, bo1, wo2t, bo2, lng, lnb, w1t, b1, w2t)


@functools.lru_cache(maxsize=None)
def _make_sc_gather(n_a, n_b, e_pad):
    info = plsc.get_sparse_core_info()
    ncores, nsub = info.num_cores, info.num_subcores
    nw = ncores * nsub
    per_w = e_pad // nw
    C = IDXW
    n_chunks = per_w // C
    NB = 4
    n_outer = n_chunks // NB
    mesh = plsc.VectorSubcoreMesh(core_axis_name="c", subcore_axis_name="s")

    @functools.partial(
        pl.kernel,
        mesh=mesh,
        compiler_params=pltpu.CompilerParams(use_tc_tiling_on_sc=False),
        out_type=jax.ShapeDtypeStruct((e_pad, EMB), jnp.bfloat16),
        scratch_types=[
            pltpu.VMEM((n_chunks, IDXW), jnp.int32),
            pltpu.VMEM((n_chunks, IDXW), jnp.int32),
        ]
        + [pltpu.VMEM((C, EMB), jnp.bfloat16)] * (2 * NB)
        + [pltpu.SemaphoreType.DMA] * (2 * NB),
    )
    def k(a_hbm, b_hbm, dst_hbm, src_hbm, m_hbm, dsti, srci, *bs):
        ras = bs[:NB]
        rbs = bs[NB : 2 * NB]
        gss = bs[2 * NB : 3 * NB]
        wss = bs[3 * NB :]
        wid = lax.axis_index("s") * ncores + lax.axis_index("c")
        base0 = wid * per_w
        irow0 = pl.multiple_of(wid * n_chunks, 8)
        pltpu.sync_copy(dst_hbm.at[pl.ds(irow0, n_chunks)], dsti)
        pltpu.sync_copy(src_hbm.at[pl.ds(irow0, n_chunks)], srci)

        def issue_gather(j, b):
            pltpu.async_copy(a_hbm.at[dsti.at[j]], ras[b], gss[b])
            pltpu.async_copy(b_hbm.at[srci.at[j]], rbs[b], gss[b])

        def wait_gather(j, b):
            pltpu.make_async_copy(a_hbm.at[dsti.at[j]], ras[b], gss[b]).wait()
            pltpu.make_async_copy(b_hbm.at[srci.at[j]], rbs[b], gss[b]).wait()

        def issue_wb(j, b):
            base = pl.multiple_of(base0 + j * C, C)
            pltpu.async_copy(ras[b], m_hbm.at[pl.ds(base, C)], wss[b])

        def wait_wb(j, b):
            base = pl.multiple_of(base0 + j * C, C)
            pltpu.make_async_copy(ras[b], m_hbm.at[pl.ds(base, C)], wss[b]).wait()

        issue_gather(0, 0)
        issue_gather(1, 1)

        def outer(jj, carry):
            for b in range(NB):
                j = jj * NB + b
                bn = (b + 2) % NB

                @pl.when(j >= 2)
                def _():
                    wait_wb(j - 2, bn)

                @pl.when(j + 2 < n_chunks)
                def _():
                    issue_gather(j + 2, bn)

                wait_gather(j, b)
                unroll = 8

                def add_rows(i8, c):
                    for r in range(unroll):
                        i = i8 * unroll + r
                        for c4 in range(EMB // (2 * LANES)):
                            sl = pl.ds(c4 * 2 * LANES, 2 * LANES)
                            ras[b][i, sl] = ras[b][i, sl] + rbs[b][i, sl]
                    return c

                lax.fori_loop(0, C // unroll, add_rows, 0)
                issue_wb(j, b)
            return carry

        lax.fori_loop(0, n_outer, outer, 0)
        wait_wb(n_chunks - 2, (n_chunks - 2) % NB)
        wait_wb(n_chunks - 1, (n_chunks - 1) % NB)

    return k


@functools.lru_cache(maxsize=None)
def _make_sc_scatter(n_nodes, e_pad):
    info = plsc.get_sparse_core_info()
    ncores, nsub = info.num_cores, info.num_subcores
    half = EMB // 2
    per_tile_e = e_pad // nsub
    C = 256
    n_chunks = per_tile_e // C
    nidx = C // IDXW
    unit = 200
    n_units = n_nodes // unit
    units_per_tile = (n_units + nsub - 1) // nsub
    mesh = plsc.VectorSubcoreMesh(core_axis_name="c", subcore_axis_name="s")

    out_sd = jax.ShapeDtypeStruct((n_nodes, half), jnp.float32)

    @functools.partial(
        pl.kernel,
        mesh=mesh,
        compiler_params=pltpu.CompilerParams(use_tc_tiling_on_sc=False),
        out_type=(out_sd, out_sd),
        scratch_types=[
            pltpu.VMEM((nidx, IDXW), jnp.int32),
            pltpu.VMEM((nidx, IDXW), jnp.int32),
            pltpu.VMEM((C, half), jnp.float32),
            pltpu.VMEM((C, half), jnp.float32),
            pltpu.VMEM((unit, half), jnp.float32),
            pltpu.MemorySpace.VMEM_SHARED((n_nodes + TRASH, half), jnp.float32),
            pltpu.SemaphoreType.DMA,
            pltpu.SemaphoreType.DMA,
            pltpu.SemaphoreType.DMA,
            pltpu.SemaphoreType.DMA,
        ],
    )
    def k(lo_hbm, hi_hbm, dst_hbm, out_lo, out_hi, idx0, idx1, rows0, rows1,
          zbuf, agg_sh, rs0, rs1, ss0, ss1):
        ibufs = (idx0, idx1)
        rbufs = (rows0, rows1)
        rsem = (rs0, rs1)
        ssem = (ss0, ss1)
        core = lax.axis_index("c")
        sub = lax.axis_index("s")

        def zb(i, c):
            for c2 in range(half // LANES):
                zbuf[i, pl.ds(c2 * LANES, LANES)] = jnp.zeros((LANES,), jnp.float32)
            return c

        lax.fori_loop(0, unit, zb, 0)

        def zz(i, c):
            u = i * nsub + sub

            @pl.when(u < n_units)
            def _():
                off = pl.multiple_of(u * unit, 8)
                pltpu.sync_copy(zbuf, agg_sh.at[pl.ds(off, unit)])

            return c

        lax.fori_loop(0, units_per_tile, zz, 0)

        @pl.when(sub == 0)
        def _():
            pltpu.sync_copy(zbuf.at[pl.ds(0, TRASH)],
                            agg_sh.at[pl.ds(n_nodes, TRASH)])

        plsc.subcore_barrier()

        base0 = sub * per_tile_e
        irow_base = sub * (per_tile_e // IDXW)

        def issue_read(j, b):
            base = pl.multiple_of(base0 + j * C, C)
            irow = pl.multiple_of(irow_base + j * nidx, nidx)
            pltpu.async_copy(dst_hbm.at[pl.ds(irow, nidx)], ibufs[b], rsem[b])

            @pl.when(core == 0)
            def _():
                pltpu.async_copy(lo_hbm.at[pl.ds(base, C)], rbufs[b], rsem[b])

            @pl.when(core == 1)
            def _():
                pltpu.async_copy(hi_hbm.at[pl.ds(base, C)], rbufs[b], rsem[b])

        def wait_read(j, b):
            base = pl.multiple_of(base0 + j * C, C)
            irow = pl.multiple_of(irow_base + j * nidx, nidx)
            pltpu.make_async_copy(
                dst_hbm.at[pl.ds(irow, nidx)], ibufs[b], rsem[b]).wait()
            pltpu.make_async_copy(
                lo_hbm.at[pl.ds(base, C)], rbufs[b], rsem[b]).wait()

        def issue_scat(j, b):
            for q in range(nidx):
                pltpu.async_copy(
                    rbufs[b].at[pl.ds(q * IDXW, IDXW)],
                    agg_sh.at[ibufs[b].at[q]], ssem[b], add=True)

        def wait_scat(j, b):
            for q in range(nidx):
                pltpu.make_async_copy(
                    rbufs[b].at[pl.ds(q * IDXW, IDXW)],
                    agg_sh.at[ibufs[b].at[q]], ssem[b]).wait()

        issue_read(0, 0)

        def outer(jj, carry):
            for b in range(2):
                j = jj * 2 + b
                wait_read(j, b)
                issue_scat(j, b)

                @pl.when(j >= 1)
                def _():
                    wait_scat(j - 1, 1 - b)

                @pl.when(j + 1 < n_chunks)
                def _():
                    issue_read(j + 1, 1 - b)

            return carry

        lax.fori_loop(0, n_chunks // 2, outer, 0)
        wait_scat(n_chunks - 1, (n_chunks - 1) % 2)
        plsc.subcore_barrier()

        def wb(i, c):
            u = i * nsub + sub

            @pl.when(u < n_units)
            def _():
                sl = pl.ds(pl.multiple_of(u * unit, 8), unit)

                @pl.when(core == 0)
                def _():
                    pltpu.sync_copy(agg_sh.at[sl], out_lo.at[sl])

                @pl.when(core == 1)
                def _():
                    pltpu.sync_copy(agg_sh.at[sl], out_hi.at[sl])

            return c

        lax.fori_loop(0, units_per_tile, wb, 0)

    return k


def _edge_stage(a_nodes, b_nodes, dst_g, src_g, dst_s, e_pad, pc):
    n_r = a_nodes.shape[0]
    n_l = b_nodes.shape[0]
    m = _make_sc_gather(n_r, n_l, e_pad)(a_nodes, b_nodes, dst_g, src_g)
    mf_lo, mf_hi = _tc_edge(
        m, pc["ln_f_g"].reshape(1, EMB), pc["ln_f_b"].reshape(1, EMB),
        pc["Wf"].T, pc["bf"].reshape(1, EMB), 4096)
    return _make_sc_scatter(n_r, e_pad)(mf_lo, mf_hi, dst_s)


def _bias_a(pc, e_bias):
    return (pc["bl"] + e_bias * pc["We"][:, 0]).reshape(1, EMB)


def kernel(constraint_features, edge_indices, edge_features, variable_features, params):
    del edge_features
    p = params
    n_c = constraint_features.shape[0]
    n_v = variable_features.shape[0]
    e = edge_indices.shape[1]
    blk = 2000
    zero64 = jnp.zeros((1, EMB), jnp.float32)

    lane_chunk = 32 * GROUP
    e_pad = ((e + lane_chunk - 1) // lane_chunk) * lane_chunk

    ei0 = edge_indices[0].astype(jnp.int32)
    ei1 = edge_indices[1].astype(jnp.int32)
    pad = e_pad - e

    def pad_to(v, fill):
        return jnp.pad(v, (0, pad), constant_values=fill).reshape(e_pad // IDXW, IDXW)

    ei0_g = pad_to(ei0, 0)
    ei1_g = pad_to(ei1, 0)
    ei0_s = pad_to(ei0, n_c)
    ei1_s = pad_to(ei1, n_v)

    e_bias = p["ln_e_b"][0]
    pvc = p["conv_vc"]
    pcv = p["conv_cv"]

    cemb, a_vc = _tc_embed(
        constraint_features,
        p["ln_c_in_g"].reshape(1, -1), p["ln_c_in_b"].reshape(1, -1),
        p["Wc1"].T, p["bc1"].reshape(1, EMB), p["Wc2"].T, p["bc2"].reshape(1, EMB),
        [(pvc["Wl"].T, _bias_a(pvc, e_bias))], blk)
    vemb, b_vc, a_cv = _tc_embed(
        variable_features,
        p["ln_v_in_g"].reshape(1, -1), p["ln_v_in_b"].reshape(1, -1),
        p["Wv1"].T, p["bv1"].reshape(1, EMB), p["Wv2"].T, p["bv2"].reshape(1, EMB),
        [(pvc["Wr"].T, zero64), (pcv["Wl"].T, _bias_a(pcv, e_bias))], blk)

    agg_lo, agg_hi = _edge_stage(a_vc, b_vc, ei0_g, ei1_g, ei0_s, e_pad, pvc)
    c2, b_cv = _tc_post(
        agg_lo, agg_hi, cemb,
        pvc["ln_post_g"].reshape(1, EMB), pvc["ln_post_b"].reshape(1, EMB),
        pvc["Wo1"].T, pvc["bo1"].reshape(1, EMB),
        pvc["Wo2"].T, pvc["bo2"].reshape(1, EMB),
        p["ln_c_g"].reshape(1, EMB), p["ln_c_b"].reshape(1, EMB),
        [(pcv["Wr"].T, zero64)], blk)

    agg_lo2, agg_hi2 = _edge_stage(a_cv, b_cv, ei1_g, ei0_g, ei1_s, e_pad, pcv)
    out = _tc_post_final(
        agg_lo2, agg_hi2, vemb,
        pcv["ln_post_g"].reshape(1, EMB), pcv["ln_post_b"].reshape(1, EMB),
        pcv["Wo1"].T, pcv["bo1"].reshape(1, EMB),
        pcv["Wo2"].T, pcv["bo2"].reshape(1, EMB),
        p["ln_v_g"].reshape(1, EMB), p["ln_v_b"].reshape(1, EMB),
        p["Wout1"].T, p["bout1"].reshape(1, EMB), p["Wout2"].T, blk)
    return out[:, 0]


# speedup vs baseline: 2.3486x; 1.0152x over previous
import functools

import jax
import jax.numpy as jnp
from jax import lax
from jax.experimental import pallas as pl
from jax.experimental.pallas import tpu as pltpu
from jax.experimental.pallas import tpu_sc as plsc

EMB = 64
LANES = 16
IDXW = 128
CHUNK = 512
GROUP = 1024
TRASH = 128

_EPS = 1e-5


def _ln_block(x, g, b):
    mu = jnp.mean(x, axis=-1, keepdims=True)
    var = jnp.mean((x - mu) * (x - mu), axis=-1, keepdims=True)
    return (x - mu) * lax.rsqrt(var + _EPS) * g + b


def _embed_body(nderiv, x_ref, lng_ref, lnb_ref, w1_ref, b1_ref, w2_ref, b2_ref,
                *rest):
    x = _ln_block(x_ref[...], lng_ref[...], lnb_ref[...])
    h = jnp.maximum(
        jnp.dot(x, w1_ref[...], preferred_element_type=jnp.float32) + b1_ref[...], 0.0)
    emb = jnp.maximum(
        jnp.dot(h, w2_ref[...], preferred_element_type=jnp.float32) + b2_ref[...], 0.0)
    wts = rest[: 2 * nderiv]
    outs = rest[2 * nderiv :]
    outs[0][...] = emb
    for i in range(nderiv):
        outs[1 + i][...] = (jnp.dot(
            emb, wts[2 * i][...], preferred_element_type=jnp.float32
        ) + wts[2 * i + 1][...]).astype(jnp.bfloat16)


def _tc_embed(x, lng, lnb, w1t, b1, w2t, b2, derivs, blk):
    n, f = x.shape
    grid = (n // blk,)
    full = lambda a: pl.BlockSpec(a.shape, lambda i: (0,) * a.ndim)
    args = [x, lng, lnb, w1t, b1, w2t, b2]
    in_specs = [
        pl.BlockSpec((blk, f), lambda i: (i, 0)),
        full(lng), full(lnb), full(w1t), full(b1), full(w2t), full(b2),
    ]
    body = functools.partial(_embed_body, len(derivs))
    for wt, bias in derivs:
        args.extend([wt, bias])
        in_specs.extend([full(wt), full(bias)])
    n_out = 1 + len(derivs)
    return pl.pallas_call(
        body,
        grid=grid,
        in_specs=in_specs,
        out_specs=[pl.BlockSpec((blk, EMB), lambda i: (i, 0))] * n_out,
        out_shape=[jax.ShapeDtypeStruct((n, EMB), jnp.float32)]
        + [jax.ShapeDtypeStruct((n, EMB), jnp.bfloat16)] * len(derivs),
    )(*args)


def _edge_body(m_ref, mmat_ref, g_ref, b_ref, wft_ref, bf_ref, lo_ref, hi_ref):
    x = m_ref[...]
    xf = x.astype(jnp.float32)
    mm = mmat_ref[...]
    mean = jnp.dot(x, mm.astype(jnp.bfloat16), preferred_element_type=jnp.float32)
    d = xf - mean
    var = jnp.dot(d * d, mm, preferred_element_type=jnp.float32)
    t = jnp.maximum(
        d * lax.rsqrt(var + _EPS) * g_ref[...] + b_ref[...], 0.0)
    mf = jnp.dot(t, wft_ref[...], preferred_element_type=jnp.float32) + bf_ref[...]
    lo_ref[...] = mf[:, : EMB // 2]
    hi_ref[...] = mf[:, EMB // 2 :]


def _tc_edge(m, mmat, g, b, wft, bf, blk):
    e = m.shape[0]
    grid = (e // blk,)
    full = lambda a: pl.BlockSpec(a.shape, lambda i: (0,) * a.ndim)
    return pl.pallas_call(
        _edge_body,
        grid=grid,
        in_specs=[
            pl.BlockSpec((blk, EMB), lambda i: (i, 0)),
            full(mmat), full(g), full(b), full(wft), full(bf),
        ],
        out_specs=[
            pl.BlockSpec((blk, EMB // 2), lambda i: (i, 0)),
            pl.BlockSpec((blk, EMB // 2), lambda i: (i, 0)),
        ],
        out_shape=[
            jax.ShapeDtypeStruct((e, EMB // 2), jnp.float32),
            jax.ShapeDtypeStruct((e, EMB // 2), jnp.float32),
        ],
    )(m, mmat, g, b, wft, bf)


def _post_node(alo_ref, ahi_ref, r_ref, png_ref, pnb_ref, wo1t_ref, bo1_ref,
               wo2t_ref, bo2_ref, lng_ref, lnb_ref):
    agg = jnp.concatenate([alo_ref[...], ahi_ref[...]], axis=-1)
    a = _ln_block(agg, png_ref[...], pnb_ref[...])
    h = jnp.concatenate([a, r_ref[...]], axis=-1)
    h = jnp.maximum(
        jnp.dot(h, wo1t_ref[...], preferred_element_type=jnp.float32) + bo1_ref[...], 0.0)
    x = jnp.dot(h, wo2t_ref[...], preferred_element_type=jnp.float32) + bo2_ref[...]
    return _ln_block(x, lng_ref[...], lnb_ref[...])


def _post_body(nderiv, *refs):
    core, rest = refs[:11], refs[11:]
    x = _post_node(*core)
    wts = rest[: 2 * nderiv]
    outs = rest[2 * nderiv :]
    outs[0][...] = x
    for i in range(nderiv):
        outs[1 + i][...] = (jnp.dot(
            x, wts[2 * i][...], preferred_element_type=jnp.float32
        ) + wts[2 * i + 1][...]).astype(jnp.bfloat16)


def _tc_post(alo, ahi, r_emb, png, pnb, wo1t, bo1, wo2t, bo2, lng, lnb,
             derivs, blk):
    n = r_emb.shape[0]
    grid = (n // blk,)
    full = lambda a: pl.BlockSpec(a.shape, lambda i: (0,) * a.ndim)
    args = [alo, ahi, r_emb, png, pnb, wo1t, bo1, wo2t, bo2, lng, lnb]
    in_specs = [
        pl.BlockSpec((blk, EMB // 2), lambda i: (i, 0)),
        pl.BlockSpec((blk, EMB // 2), lambda i: (i, 0)),
        pl.BlockSpec((blk, EMB), lambda i: (i, 0)),
        full(png), full(pnb), full(wo1t), full(bo1),
        full(wo2t), full(bo2), full(lng), full(lnb),
    ]
    for wt, bias in derivs:
        args.extend([wt, bias])
        in_specs.extend([full(wt), full(bias)])
    n_out = 1 + len(derivs)
    return pl.pallas_call(
        functools.partial(_post_body, len(derivs)),
        grid=grid,
        in_specs=in_specs,
        out_specs=[pl.BlockSpec((blk, EMB), lambda i: (i, 0))] * n_out,
        out_shape=[jax.ShapeDtypeStruct((n, EMB), jnp.float32)]
        + [jax.ShapeDtypeStruct((n, EMB), jnp.bfloat16)] * len(derivs),
    )(*args)


def _post_final_body(alo_ref, ahi_ref, r_ref, png_ref, pnb_ref, wo1t_ref,
                     bo1_ref, wo2t_ref, bo2_ref, lng_ref, lnb_ref,
                     w1t_ref, b1_ref, w2t_ref, o_ref):
    x = _post_node(alo_ref, ahi_ref, r_ref, png_ref, pnb_ref, wo1t_ref,
                   bo1_ref, wo2t_ref, bo2_ref, lng_ref, lnb_ref)
    h = jnp.maximum(
        jnp.dot(x, w1t_ref[...], preferred_element_type=jnp.float32)
        + b1_ref[...], 0.0)
    o_ref[...] = jnp.dot(h, w2t_ref[...], preferred_element_type=jnp.float32)


def _tc_post_final(alo, ahi, r_emb, png, pnb, wo1t, bo1, wo2t, bo2, lng, lnb,
                   w1t, b1, w2t, blk):
    n = r_emb.shape[0]
    grid = (n // blk,)
    full = lambda a: pl.BlockSpec(a.shape, lambda i: (0,) * a.ndim)
    return pl.pallas_call(
        _post_final_body,
        grid=grid,
        in_specs=[
            pl.BlockSpec((blk, EMB // 2), lambda i: (i, 0)),
            pl.BlockSpec((blk, EMB // 2), lambda i: (i, 0)),
            pl.BlockSpec((blk, EMB), lambda i: (i, 0)),
            full(png), full(pnb), full(wo1t), full(bo1),
            full(wo2t), full(bo2), full(lng), full(lnb),
            full(w1t), full(b1), full(w2t),
        ],
        out_specs=pl.BlockSpec((blk, 1), lambda i: (i, 0)),
        out_shape=jax.ShapeDtypeStruct((n, 1), jnp.float32),
    )(alo, ahi, r_emb, png, pnb, wo1t, bo1, wo2t, bo2, lng, lnb, w1t, b1, w2t)


@functools.lru_cache(maxsize=None)
def _make_sc_gather(n_a, n_b, e_pad):
    info = plsc.get_sparse_core_info()
    ncores, nsub = info.num_cores, info.num_subcores
    nw = ncores * nsub
    per_w = e_pad // nw
    C = IDXW
    n_chunks = per_w // C
    NB = 4
    n_outer = n_chunks // NB
    mesh = plsc.VectorSubcoreMesh(core_axis_name="c", subcore_axis_name="s")

    @functools.partial(
        pl.kernel,
        mesh=mesh,
        compiler_params=pltpu.CompilerParams(use_tc_tiling_on_sc=False),
        out_type=jax.ShapeDtypeStruct((e_pad, EMB), jnp.bfloat16),
        scratch_types=[
            pltpu.VMEM((n_chunks, IDXW), jnp.int32),
            pltpu.VMEM((n_chunks, IDXW), jnp.int32),
        ]
        + [pltpu.VMEM((C, EMB), jnp.bfloat16)] * (2 * NB)
        + [pltpu.SemaphoreType.DMA] * (2 * NB),
    )
    def k(a_hbm, b_hbm, dst_hbm, src_hbm, m_hbm, dsti, srci, *bs):
        ras = bs[:NB]
        rbs = bs[NB : 2 * NB]
        gss = bs[2 * NB : 3 * NB]
        wss = bs[3 * NB :]
        wid = lax.axis_index("s") * ncores + lax.axis_index("c")
        base0 = wid * per_w
        irow0 = pl.multiple_of(wid * n_chunks, 8)
        pltpu.sync_copy(dst_hbm.at[pl.ds(irow0, n_chunks)], dsti)
        pltpu.sync_copy(src_hbm.at[pl.ds(irow0, n_chunks)], srci)

        def issue_gather(j, b):
            pltpu.async_copy(a_hbm.at[dsti.at[j]], ras[b], gss[b])
            pltpu.async_copy(b_hbm.at[srci.at[j]], rbs[b], gss[b])

        def wait_gather(j, b):
            pltpu.make_async_copy(a_hbm.at[dsti.at[j]], ras[b], gss[b]).wait()
            pltpu.make_async_copy(b_hbm.at[srci.at[j]], rbs[b], gss[b]).wait()

        def issue_wb(j, b):
            base = pl.multiple_of(base0 + j * C, C)
            pltpu.async_copy(ras[b], m_hbm.at[pl.ds(base, C)], wss[b])

        def wait_wb(j, b):
            base = pl.multiple_of(base0 + j * C, C)
            pltpu.make_async_copy(ras[b], m_hbm.at[pl.ds(base, C)], wss[b]).wait()

        issue_gather(0, 0)
        issue_gather(1, 1)

        def outer(jj, carry):
            for b in range(NB):
                j = jj * NB + b
                bn = (b + 2) % NB

                @pl.when(j >= 2)
                def _():
                    wait_wb(j - 2, bn)

                @pl.when(j + 2 < n_chunks)
                def _():
                    issue_gather(j + 2, bn)

                wait_gather(j, b)
                unroll = 8

                def add_rows(i8, c):
                    for r in range(unroll):
                        i = i8 * unroll + r
                        for c4 in range(EMB // (2 * LANES)):
                            sl = pl.ds(c4 * 2 * LANES, 2 * LANES)
                            ras[b][i, sl] = ras[b][i, sl] + rbs[b][i, sl]
                    return c

                lax.fori_loop(0, C // unroll, add_rows, 0)
                issue_wb(j, b)
            return carry

        lax.fori_loop(0, n_outer, outer, 0)
        wait_wb(n_chunks - 2, (n_chunks - 2) % NB)
        wait_wb(n_chunks - 1, (n_chunks - 1) % NB)

    return k


@functools.lru_cache(maxsize=None)
def _make_sc_scatter(n_nodes, e_pad):
    info = plsc.get_sparse_core_info()
    ncores, nsub = info.num_cores, info.num_subcores
    half = EMB // 2
    per_tile_e = e_pad // nsub
    C = 256
    n_chunks = per_tile_e // C
    nidx = C // IDXW
    unit = 200
    n_units = n_nodes // unit
    units_per_tile = (n_units + nsub - 1) // nsub
    mesh = plsc.VectorSubcoreMesh(core_axis_name="c", subcore_axis_name="s")

    out_sd = jax.ShapeDtypeStruct((n_nodes, half), jnp.float32)

    @functools.partial(
        pl.kernel,
        mesh=mesh,
        compiler_params=pltpu.CompilerParams(use_tc_tiling_on_sc=False),
        out_type=(out_sd, out_sd),
        scratch_types=[
            pltpu.VMEM((nidx, IDXW), jnp.int32),
            pltpu.VMEM((nidx, IDXW), jnp.int32),
            pltpu.VMEM((C, half), jnp.float32),
            pltpu.VMEM((C, half), jnp.float32),
            pltpu.VMEM((unit, half), jnp.float32),
            pltpu.MemorySpace.VMEM_SHARED((n_nodes + TRASH, half), jnp.float32),
            pltpu.SemaphoreType.DMA,
            pltpu.SemaphoreType.DMA,
            pltpu.SemaphoreType.DMA,
            pltpu.SemaphoreType.DMA,
        ],
    )
    def k(lo_hbm, hi_hbm, dst_hbm, out_lo, out_hi, idx0, idx1, rows0, rows1,
          zbuf, agg_sh, rs0, rs1, ss0, ss1):
        ibufs = (idx0, idx1)
        rbufs = (rows0, rows1)
        rsem = (rs0, rs1)
        ssem = (ss0, ss1)
        core = lax.axis_index("c")
        sub = lax.axis_index("s")

        def zb(i, c):
            for c2 in range(half // LANES):
                zbuf[i, pl.ds(c2 * LANES, LANES)] = jnp.zeros((LANES,), jnp.float32)
            return c

        lax.fori_loop(0, unit, zb, 0)

        def zz(i, c):
            u = i * nsub + sub

            @pl.when(u < n_units)
            def _():
                off = pl.multiple_of(u * unit, 8)
                pltpu.sync_copy(zbuf, agg_sh.at[pl.ds(off, unit)])

            return c

        lax.fori_loop(0, units_per_tile, zz, 0)

        @pl.when(sub == 0)
        def _():
            pltpu.sync_copy(zbuf.at[pl.ds(0, TRASH)],
                            agg_sh.at[pl.ds(n_nodes, TRASH)])

        plsc.subcore_barrier()

        base0 = sub * per_tile_e
        irow_base = sub * (per_tile_e // IDXW)

        def issue_read(j, b):
            base = pl.multiple_of(base0 + j * C, C)
            irow = pl.multiple_of(irow_base + j * nidx, nidx)
            pltpu.async_copy(dst_hbm.at[pl.ds(irow, nidx)], ibufs[b], rsem[b])

            @pl.when(core == 0)
            def _():
                pltpu.async_copy(lo_hbm.at[pl.ds(base, C)], rbufs[b], rsem[b])

            @pl.when(core == 1)
            def _():
                pltpu.async_copy(hi_hbm.at[pl.ds(base, C)], rbufs[b], rsem[b])

        def wait_read(j, b):
            base = pl.multiple_of(base0 + j * C, C)
            irow = pl.multiple_of(irow_base + j * nidx, nidx)
            pltpu.make_async_copy(
                dst_hbm.at[pl.ds(irow, nidx)], ibufs[b], rsem[b]).wait()
            pltpu.make_async_copy(
                lo_hbm.at[pl.ds(base, C)], rbufs[b], rsem[b]).wait()

        def issue_scat(j, b):
            for q in range(nidx):
                pltpu.async_copy(
                    rbufs[b].at[pl.ds(q * IDXW, IDXW)],
                    agg_sh.at[ibufs[b].at[q]], ssem[b], add=True)

        def wait_scat(j, b):
            for q in range(nidx):
                pltpu.make_async_copy(
                    rbufs[b].at[pl.ds(q * IDXW, IDXW)],
                    agg_sh.at[ibufs[b].at[q]], ssem[b]).wait()

        issue_read(0, 0)

        def outer(jj, carry):
            for b in range(2):
                j = jj * 2 + b
                wait_read(j, b)
                issue_scat(j, b)

                @pl.when(j >= 1)
                def _():
                    wait_scat(j - 1, 1 - b)

                @pl.when(j + 1 < n_chunks)
                def _():
                    issue_read(j + 1, 1 - b)

            return carry

        lax.fori_loop(0, n_chunks // 2, outer, 0)
        wait_scat(n_chunks - 1, (n_chunks - 1) % 2)
        plsc.subcore_barrier()

        def wb(i, c):
            u = i * nsub + sub

            @pl.when(u < n_units)
            def _():
                sl = pl.ds(pl.multiple_of(u * unit, 8), unit)

                @pl.when(core == 0)
                def _():
                    pltpu.sync_copy(agg_sh.at[sl], out_lo.at[sl])

                @pl.when(core == 1)
                def _():
                    pltpu.sync_copy(agg_sh.at[sl], out_hi.at[sl])

            return c

        lax.fori_loop(0, units_per_tile, wb, 0)

    return k


def _edge_stage(a_nodes, b_nodes, dst_g, src_g, dst_s, e_pad, pc):
    n_r = a_nodes.shape[0]
    n_l = b_nodes.shape[0]
    m = _make_sc_gather(n_r, n_l, e_pad)(a_nodes, b_nodes, dst_g, src_g)
    mmat = jnp.full((EMB, EMB), 1.0 / EMB, jnp.float32)
    mf_lo, mf_hi = _tc_edge(
        m, mmat, pc["ln_f_g"].reshape(1, EMB), pc["ln_f_b"].reshape(1, EMB),
        pc["Wf"].T, pc["bf"].reshape(1, EMB), 4096)
    return _make_sc_scatter(n_r, e_pad)(mf_lo, mf_hi, dst_s)


def _bias_a(pc, e_bias):
    return (pc["bl"] + e_bias * pc["We"][:, 0]).reshape(1, EMB)


def kernel(constraint_features, edge_indices, edge_features, variable_features, params):
    del edge_features
    p = params
    n_c = constraint_features.shape[0]
    n_v = variable_features.shape[0]
    e = edge_indices.shape[1]
    blk = 2000
    zero64 = jnp.zeros((1, EMB), jnp.float32)

    lane_chunk = 32 * GROUP
    e_pad = ((e + lane_chunk - 1) // lane_chunk) * lane_chunk

    ei0 = edge_indices[0].astype(jnp.int32)
    ei1 = edge_indices[1].astype(jnp.int32)
    pad = e_pad - e

    def pad_to(v, fill):
        return jnp.pad(v, (0, pad), constant_values=fill).reshape(e_pad // IDXW, IDXW)

    ei0_g = pad_to(ei0, 0)
    ei1_g = pad_to(ei1, 0)
    ei0_s = pad_to(ei0, n_c)
    ei1_s = pad_to(ei1, n_v)

    e_bias = p["ln_e_b"][0]
    pvc = p["conv_vc"]
    pcv = p["conv_cv"]

    cemb, a_vc = _tc_embed(
        constraint_features,
        p["ln_c_in_g"].reshape(1, -1), p["ln_c_in_b"].reshape(1, -1),
        p["Wc1"].T, p["bc1"].reshape(1, EMB), p["Wc2"].T, p["bc2"].reshape(1, EMB),
        [(pvc["Wl"].T, _bias_a(pvc, e_bias))], blk)
    vemb, b_vc, a_cv = _tc_embed(
        variable_features,
        p["ln_v_in_g"].reshape(1, -1), p["ln_v_in_b"].reshape(1, -1),
        p["Wv1"].T, p["bv1"].reshape(1, EMB), p["Wv2"].T, p["bv2"].reshape(1, EMB),
        [(pvc["Wr"].T, zero64), (pcv["Wl"].T, _bias_a(pcv, e_bias))], blk)

    agg_lo, agg_hi = _edge_stage(a_vc, b_vc, ei0_g, ei1_g, ei0_s, e_pad, pvc)
    c2, b_cv = _tc_post(
        agg_lo, agg_hi, cemb,
        pvc["ln_post_g"].reshape(1, EMB), pvc["ln_post_b"].reshape(1, EMB),
        pvc["Wo1"].T, pvc["bo1"].reshape(1, EMB),
        pvc["Wo2"].T, pvc["bo2"].reshape(1, EMB),
        p["ln_c_g"].reshape(1, EMB), p["ln_c_b"].reshape(1, EMB),
        [(pcv["Wr"].T, zero64)], blk)

    agg_lo2, agg_hi2 = _edge_stage(a_cv, b_cv, ei1_g, ei0_g, ei1_s, e_pad, pcv)
    out = _tc_post_final(
        agg_lo2, agg_hi2, vemb,
        pcv["ln_post_g"].reshape(1, EMB), pcv["ln_post_b"].reshape(1, EMB),
        pcv["Wo1"].T, pcv["bo1"].reshape(1, EMB),
        pcv["Wo2"].T, pcv["bo2"].reshape(1, EMB),
        p["ln_v_g"].reshape(1, EMB), p["ln_v_b"].reshape(1, EMB),
        p["Wout1"].T, p["bout1"].reshape(1, EMB), p["Wout2"].T, blk)
    return out[:, 0]


# speedup vs baseline: 2.3492x; 1.0003x over previous
import functools

import jax
import jax.numpy as jnp
from jax import lax
from jax.experimental import pallas as pl
from jax.experimental.pallas import tpu as pltpu
from jax.experimental.pallas import tpu_sc as plsc

EMB = 64
LANES = 16
IDXW = 128
CHUNK = 512
GROUP = 1024
TRASH = 128

_EPS = 1e-5


def _ln_block(x, g, b):
    mu = jnp.mean(x, axis=-1, keepdims=True)
    var = jnp.mean((x - mu) * (x - mu), axis=-1, keepdims=True)
    return (x - mu) * lax.rsqrt(var + _EPS) * g + b


def _embed_body(nderiv, x_ref, lng_ref, lnb_ref, w1_ref, b1_ref, w2_ref, b2_ref,
                *rest):
    x = _ln_block(x_ref[...], lng_ref[...], lnb_ref[...])
    h = jnp.maximum(
        jnp.dot(x, w1_ref[...], preferred_element_type=jnp.float32) + b1_ref[...], 0.0)
    emb = jnp.maximum(
        jnp.dot(h, w2_ref[...], preferred_element_type=jnp.float32) + b2_ref[...], 0.0)
    wts = rest[: 2 * nderiv]
    outs = rest[2 * nderiv :]
    outs[0][...] = emb
    for i in range(nderiv):
        outs[1 + i][...] = (jnp.dot(
            emb, wts[2 * i][...], preferred_element_type=jnp.float32
        ) + wts[2 * i + 1][...]).astype(jnp.bfloat16)


def _tc_embed(x, lng, lnb, w1t, b1, w2t, b2, derivs, blk):
    n, f = x.shape
    grid = (n // blk,)
    full = lambda a: pl.BlockSpec(a.shape, lambda i: (0,) * a.ndim)
    args = [x, lng, lnb, w1t, b1, w2t, b2]
    in_specs = [
        pl.BlockSpec((blk, f), lambda i: (i, 0)),
        full(lng), full(lnb), full(w1t), full(b1), full(w2t), full(b2),
    ]
    body = functools.partial(_embed_body, len(derivs))
    for wt, bias in derivs:
        args.extend([wt, bias])
        in_specs.extend([full(wt), full(bias)])
    n_out = 1 + len(derivs)
    return pl.pallas_call(
        body,
        grid=grid,
        in_specs=in_specs,
        out_specs=[pl.BlockSpec((blk, EMB), lambda i: (i, 0))] * n_out,
        out_shape=[jax.ShapeDtypeStruct((n, EMB), jnp.float32)]
        + [jax.ShapeDtypeStruct((n, EMB), jnp.bfloat16)] * len(derivs),
    )(*args)


def _edge_body(m_ref, mmat_ref, g_ref, b_ref, wft_ref, bf_ref, lo_ref, hi_ref):
    x = m_ref[...]
    xf = x.astype(jnp.float32)
    mm = mmat_ref[...]
    mean = jnp.dot(x, mm.astype(jnp.bfloat16), preferred_element_type=jnp.float32)
    d = xf - mean
    var = jnp.dot(d * d, mm, preferred_element_type=jnp.float32)
    t = jnp.maximum(
        d * lax.rsqrt(var + _EPS) * g_ref[...] + b_ref[...], 0.0)
    mf = jnp.dot(t, wft_ref[...], preferred_element_type=jnp.float32) + bf_ref[...]
    lo_ref[...] = mf[:, : EMB // 2]
    hi_ref[...] = mf[:, EMB // 2 :]


def _tc_edge(m, mmat, g, b, wft, bf, blk):
    e = m.shape[0]
    grid = (e // blk,)
    full = lambda a: pl.BlockSpec(a.shape, lambda i: (0,) * a.ndim)
    return pl.pallas_call(
        _edge_body,
        grid=grid,
        in_specs=[
            pl.BlockSpec((blk, EMB), lambda i: (i, 0)),
            full(mmat), full(g), full(b), full(wft), full(bf),
        ],
        out_specs=[
            pl.BlockSpec((blk, EMB // 2), lambda i: (i, 0)),
            pl.BlockSpec((blk, EMB // 2), lambda i: (i, 0)),
        ],
        out_shape=[
            jax.ShapeDtypeStruct((e, EMB // 2), jnp.float32),
            jax.ShapeDtypeStruct((e, EMB // 2), jnp.float32),
        ],
    )(m, mmat, g, b, wft, bf)


def _post_node(alo_ref, ahi_ref, r_ref, png_ref, pnb_ref, wo1t_ref, bo1_ref,
               wo2t_ref, bo2_ref, lng_ref, lnb_ref):
    agg = jnp.concatenate([alo_ref[...], ahi_ref[...]], axis=-1)
    a = _ln_block(agg, png_ref[...], pnb_ref[...])
    h = jnp.concatenate([a, r_ref[...]], axis=-1)
    h = jnp.maximum(
        jnp.dot(h, wo1t_ref[...], preferred_element_type=jnp.float32) + bo1_ref[...], 0.0)
    x = jnp.dot(h, wo2t_ref[...], preferred_element_type=jnp.float32) + bo2_ref[...]
    return _ln_block(x, lng_ref[...], lnb_ref[...])


def _post_body(nderiv, *refs):
    core, rest = refs[:11], refs[11:]
    x = _post_node(*core)
    wts = rest[: 2 * nderiv]
    outs = rest[2 * nderiv :]
    outs[0][...] = x
    for i in range(nderiv):
        outs[1 + i][...] = (jnp.dot(
            x, wts[2 * i][...], preferred_element_type=jnp.float32
        ) + wts[2 * i + 1][...]).astype(jnp.bfloat16)


def _tc_post(alo, ahi, r_emb, png, pnb, wo1t, bo1, wo2t, bo2, lng, lnb,
             derivs, blk):
    n = r_emb.shape[0]
    grid = (n // blk,)
    full = lambda a: pl.BlockSpec(a.shape, lambda i: (0,) * a.ndim)
    args = [alo, ahi, r_emb, png, pnb, wo1t, bo1, wo2t, bo2, lng, lnb]
    in_specs = [
        pl.BlockSpec((blk, EMB // 2), lambda i: (i, 0)),
        pl.BlockSpec((blk, EMB // 2), lambda i: (i, 0)),
        pl.BlockSpec((blk, EMB), lambda i: (i, 0)),
        full(png), full(pnb), full(wo1t), full(bo1),
        full(wo2t), full(bo2), full(lng), full(lnb),
    ]
    for wt, bias in derivs:
        args.extend([wt, bias])
        in_specs.extend([full(wt), full(bias)])
    n_out = 1 + len(derivs)
    return pl.pallas_call(
        functools.partial(_post_body, len(derivs)),
        grid=grid,
        in_specs=in_specs,
        out_specs=[pl.BlockSpec((blk, EMB), lambda i: (i, 0))] * n_out,
        out_shape=[jax.ShapeDtypeStruct((n, EMB), jnp.float32)]
        + [jax.ShapeDtypeStruct((n, EMB), jnp.bfloat16)] * len(derivs),
    )(*args)


def _post_final_body(alo_ref, ahi_ref, r_ref, png_ref, pnb_ref, wo1t_ref,
                     bo1_ref, wo2t_ref, bo2_ref, lng_ref, lnb_ref,
                     w1t_ref, b1_ref, w2t_ref, o_ref):
    x = _post_node(alo_ref, ahi_ref, r_ref, png_ref, pnb_ref, wo1t_ref,
                   bo1_ref, wo2t_ref, bo2_ref, lng_ref, lnb_ref)
    h = jnp.maximum(
        jnp.dot(x, w1t_ref[...], preferred_element_type=jnp.float32)
        + b1_ref[...], 0.0)
    o_ref[...] = jnp.dot(h, w2t_ref[...], preferred_element_type=jnp.float32)


def _tc_post_final(alo, ahi, r_emb, png, pnb, wo1t, bo1, wo2t, bo2, lng, lnb,
                   w1t, b1, w2t, blk):
    n = r_emb.shape[0]
    grid = (n // blk,)
    full = lambda a: pl.BlockSpec(a.shape, lambda i: (0,) * a.ndim)
    return pl.pallas_call(
        _post_final_body,
        grid=grid,
        in_specs=[
            pl.BlockSpec((blk, EMB // 2), lambda i: (i, 0)),
            pl.BlockSpec((blk, EMB // 2), lambda i: (i, 0)),
            pl.BlockSpec((blk, EMB), lambda i: (i, 0)),
            full(png), full(pnb), full(wo1t), full(bo1),
            full(wo2t), full(bo2), full(lng), full(lnb),
            full(w1t), full(b1), full(w2t),
        ],
        out_specs=pl.BlockSpec((blk, 1), lambda i: (i, 0)),
        out_shape=jax.ShapeDtypeStruct((n, 1), jnp.float32),
    )(alo, ahi, r_emb, png, pnb, wo1t, bo1, wo2t, bo2, lng, lnb, w1t, b1, w2t)


@functools.lru_cache(maxsize=None)
def _make_sc_gather(n_a, n_b, e_pad):
    info = plsc.get_sparse_core_info()
    ncores, nsub = info.num_cores, info.num_subcores
    nw = ncores * nsub
    per_w = e_pad // nw
    C = IDXW
    n_chunks = per_w // C
    NB = 4
    n_outer = n_chunks // NB
    mesh = plsc.VectorSubcoreMesh(core_axis_name="c", subcore_axis_name="s")

    @functools.partial(
        pl.kernel,
        mesh=mesh,
        compiler_params=pltpu.CompilerParams(use_tc_tiling_on_sc=False),
        out_type=jax.ShapeDtypeStruct((e_pad, EMB), jnp.bfloat16),
        scratch_types=[
            pltpu.VMEM((n_chunks, IDXW), jnp.int32),
            pltpu.VMEM((n_chunks, IDXW), jnp.int32),
        ]
        + [pltpu.VMEM((C, EMB), jnp.bfloat16)] * (2 * NB)
        + [pltpu.SemaphoreType.DMA] * (2 * NB),
    )
    def k(a_hbm, b_hbm, dst_hbm, src_hbm, m_hbm, dsti, srci, *bs):
        ras = bs[:NB]
        rbs = bs[NB : 2 * NB]
        gss = bs[2 * NB : 3 * NB]
        wss = bs[3 * NB :]
        wid = lax.axis_index("s") * ncores + lax.axis_index("c")
        base0 = wid * per_w
        irow0 = pl.multiple_of(wid * n_chunks, 8)
        pltpu.sync_copy(dst_hbm.at[pl.ds(irow0, n_chunks)], dsti)
        pltpu.sync_copy(src_hbm.at[pl.ds(irow0, n_chunks)], srci)

        def issue_gather(j, b):
            pltpu.async_copy(a_hbm.at[dsti.at[j]], ras[b], gss[b])
            pltpu.async_copy(b_hbm.at[srci.at[j]], rbs[b], gss[b])

        def wait_gather(j, b):
            pltpu.make_async_copy(a_hbm.at[dsti.at[j]], ras[b], gss[b]).wait()
            pltpu.make_async_copy(b_hbm.at[srci.at[j]], rbs[b], gss[b]).wait()

        def issue_wb(j, b):
            base = pl.multiple_of(base0 + j * C, C)
            pltpu.async_copy(ras[b], m_hbm.at[pl.ds(base, C)], wss[b])

        def wait_wb(j, b):
            base = pl.multiple_of(base0 + j * C, C)
            pltpu.make_async_copy(ras[b], m_hbm.at[pl.ds(base, C)], wss[b]).wait()

        issue_gather(0, 0)
        issue_gather(1, 1)

        def outer(jj, carry):
            for b in range(NB):
                j = jj * NB + b
                bn = (b + 2) % NB

                @pl.when(j >= 2)
                def _():
                    wait_wb(j - 2, bn)

                @pl.when(j + 2 < n_chunks)
                def _():
                    issue_gather(j + 2, bn)

                wait_gather(j, b)
                unroll = 8

                def add_rows(i8, c):
                    for r in range(unroll):
                        i = i8 * unroll + r
                        for c4 in range(EMB // (2 * LANES)):
                            sl = pl.ds(c4 * 2 * LANES, 2 * LANES)
                            ras[b][i, sl] = ras[b][i, sl] + rbs[b][i, sl]
                    return c

                lax.fori_loop(0, C // unroll, add_rows, 0)
                issue_wb(j, b)
            return carry

        lax.fori_loop(0, n_outer, outer, 0)
        wait_wb(n_chunks - 2, (n_chunks - 2) % NB)
        wait_wb(n_chunks - 1, (n_chunks - 1) % NB)

    return k


@functools.lru_cache(maxsize=None)
def _make_sc_scatter(n_nodes, e_pad):
    info = plsc.get_sparse_core_info()
    ncores, nsub = info.num_cores, info.num_subcores
    half = EMB // 2
    per_tile_e = e_pad // nsub
    C = 256
    n_chunks = per_tile_e // C
    nidx = C // IDXW
    unit = 200
    n_units = n_nodes // unit
    units_per_tile = (n_units + nsub - 1) // nsub
    mesh = plsc.VectorSubcoreMesh(core_axis_name="c", subcore_axis_name="s")

    out_sd = jax.ShapeDtypeStruct((n_nodes, half), jnp.float32)

    @functools.partial(
        pl.kernel,
        mesh=mesh,
        compiler_params=pltpu.CompilerParams(use_tc_tiling_on_sc=False),
        out_type=(out_sd, out_sd),
        scratch_types=[
            pltpu.VMEM((8, IDXW), jnp.int32),
            pltpu.VMEM((8, IDXW), jnp.int32),
            pltpu.VMEM((C, half), jnp.float32),
            pltpu.VMEM((C, half), jnp.float32),
            pltpu.VMEM((unit, half), jnp.float32),
            pltpu.MemorySpace.VMEM_SHARED((n_nodes + TRASH, half), jnp.float32),
            pltpu.SemaphoreType.DMA,
            pltpu.SemaphoreType.DMA,
            pltpu.SemaphoreType.DMA,
            pltpu.SemaphoreType.DMA,
        ],
    )
    def k(lo_hbm, hi_hbm, dst_hbm, out_lo, out_hi, idx0, idx1, rows0, rows1,
          zbuf, agg_sh, rs0, rs1, ss0, ss1):
        ibufs = (idx0, idx1)
        rbufs = (rows0, rows1)
        rsem = (rs0, rs1)
        ssem = (ss0, ss1)
        core = lax.axis_index("c")
        sub = lax.axis_index("s")

        def zb(i, c):
            for c2 in range(half // LANES):
                zbuf[i, pl.ds(c2 * LANES, LANES)] = jnp.zeros((LANES,), jnp.float32)
            return c

        lax.fori_loop(0, unit, zb, 0)

        def zz(i, c):
            u = i * nsub + sub

            @pl.when(u < n_units)
            def _():
                off = pl.multiple_of(u * unit, 8)
                pltpu.sync_copy(zbuf, agg_sh.at[pl.ds(off, unit)])

            return c

        lax.fori_loop(0, units_per_tile, zz, 0)

        @pl.when(sub == 0)
        def _():
            pltpu.sync_copy(zbuf.at[pl.ds(0, TRASH)],
                            agg_sh.at[pl.ds(n_nodes, TRASH)])

        plsc.subcore_barrier()

        base0 = sub * per_tile_e
        irow_base = sub * (per_tile_e // IDXW)

        def issue_read(j, b):
            base = pl.multiple_of(base0 + j * C, C)

            @pl.when(core == 0)
            def _():
                pltpu.async_copy(lo_hbm.at[pl.ds(base, C)], rbufs[b], rsem[b])

            @pl.when(core == 1)
            def _():
                pltpu.async_copy(hi_hbm.at[pl.ds(base, C)], rbufs[b], rsem[b])

        def wait_read(j, b):
            base = pl.multiple_of(base0 + j * C, C)
            pltpu.make_async_copy(
                lo_hbm.at[pl.ds(base, C)], rbufs[b], rsem[b]).wait()

        def load_idx_group(g, ib):
            irow = pl.multiple_of(irow_base + g * 8, 8)
            pltpu.sync_copy(dst_hbm.at[pl.ds(irow, 8)], ibufs[ib])

        def issue_scat(j, b, ib, krow):
            for q in range(nidx):
                pltpu.async_copy(
                    rbufs[b].at[pl.ds(q * IDXW, IDXW)],
                    agg_sh.at[ibufs[ib].at[krow * nidx + q]], ssem[b], add=True)

        def wait_scat(j, b, ib, krow):
            for q in range(nidx):
                pltpu.make_async_copy(
                    rbufs[b].at[pl.ds(q * IDXW, IDXW)],
                    agg_sh.at[ibufs[ib].at[krow * nidx + q]], ssem[b]).wait()

        load_idx_group(0, 0)
        issue_read(0, 0)

        def outer(it, carry):
            for k in range(8):
                j = it * 8 + k
                b = k % 2
                ib = k // 4
                if k == 0:
                    @pl.when(it > 0)
                    def _():
                        load_idx_group(it * 2, 0)
                if k == 4:
                    load_idx_group(it * 2 + 1, 1)
                wait_read(j, b)
                issue_scat(j, b, ib, k % 4)

                @pl.when(j >= 1)
                def _():
                    wait_scat(j - 1, 1 - b, (k - 1) % 8 // 4, (k - 1) % 4)

                @pl.when(j + 1 < n_chunks)
                def _():
                    issue_read(j + 1, 1 - b)

            return carry

        lax.fori_loop(0, n_chunks // 8, outer, 0)
        wait_scat(n_chunks - 1, 1, 1, 3)
        plsc.subcore_barrier()

        def wb(i, c):
            u = i * nsub + sub

            @pl.when(u < n_units)
            def _():
                sl = pl.ds(pl.multiple_of(u * unit, 8), unit)

                @pl.when(core == 0)
                def _():
                    pltpu.sync_copy(agg_sh.at[sl], out_lo.at[sl])

                @pl.when(core == 1)
                def _():
                    pltpu.sync_copy(agg_sh.at[sl], out_hi.at[sl])

            return c

        lax.fori_loop(0, units_per_tile, wb, 0)

    return k


def _edge_stage(a_nodes, b_nodes, dst_g, src_g, dst_s, e_pad, pc):
    n_r = a_nodes.shape[0]
    n_l = b_nodes.shape[0]
    m = _make_sc_gather(n_r, n_l, e_pad)(a_nodes, b_nodes, dst_g, src_g)
    mmat = jnp.full((EMB, EMB), 1.0 / EMB, jnp.float32)
    mf_lo, mf_hi = _tc_edge(
        m, mmat, pc["ln_f_g"].reshape(1, EMB), pc["ln_f_b"].reshape(1, EMB),
        pc["Wf"].T, pc["bf"].reshape(1, EMB), 4096)
    return _make_sc_scatter(n_r, e_pad)(mf_lo, mf_hi, dst_s)


def _bias_a(pc, e_bias):
    return (pc["bl"] + e_bias * pc["We"][:, 0]).reshape(1, EMB)


def kernel(constraint_features, edge_indices, edge_features, variable_features, params):
    del edge_features
    p = params
    n_c = constraint_features.shape[0]
    n_v = variable_features.shape[0]
    e = edge_indices.shape[1]
    blk = 2000
    zero64 = jnp.zeros((1, EMB), jnp.float32)

    lane_chunk = 32 * GROUP
    e_pad = ((e + lane_chunk - 1) // lane_chunk) * lane_chunk

    ei0 = edge_indices[0].astype(jnp.int32)
    ei1 = edge_indices[1].astype(jnp.int32)
    pad = e_pad - e

    def pad_to(v, fill):
        return jnp.pad(v, (0, pad), constant_values=fill).reshape(e_pad // IDXW, IDXW)

    ei0_g = pad_to(ei0, 0)
    ei1_g = pad_to(ei1, 0)
    ei0_s = pad_to(ei0, n_c)
    ei1_s = pad_to(ei1, n_v)

    e_bias = p["ln_e_b"][0]
    pvc = p["conv_vc"]
    pcv = p["conv_cv"]

    cemb, a_vc = _tc_embed(
        constraint_features,
        p["ln_c_in_g"].reshape(1, -1), p["ln_c_in_b"].reshape(1, -1),
        p["Wc1"].T, p["bc1"].reshape(1, EMB), p["Wc2"].T, p["bc2"].reshape(1, EMB),
        [(pvc["Wl"].T, _bias_a(pvc, e_bias))], blk)
    vemb, b_vc, a_cv = _tc_embed(
        variable_features,
        p["ln_v_in_g"].reshape(1, -1), p["ln_v_in_b"].reshape(1, -1),
        p["Wv1"].T, p["bv1"].reshape(1, EMB), p["Wv2"].T, p["bv2"].reshape(1, EMB),
        [(pvc["Wr"].T, zero64), (pcv["Wl"].T, _bias_a(pcv, e_bias))], blk)

    agg_lo, agg_hi = _edge_stage(a_vc, b_vc, ei0_g, ei1_g, ei0_s, e_pad, pvc)
    c2, b_cv = _tc_post(
        agg_lo, agg_hi, cemb,
        pvc["ln_post_g"].reshape(1, EMB), pvc["ln_post_b"].reshape(1, EMB),
        pvc["Wo1"].T, pvc["bo1"].reshape(1, EMB),
        pvc["Wo2"].T, pvc["bo2"].reshape(1, EMB),
        p["ln_c_g"].reshape(1, EMB), p["ln_c_b"].reshape(1, EMB),
        [(pcv["Wr"].T, zero64)], blk)

    agg_lo2, agg_hi2 = _edge_stage(a_cv, b_cv, ei1_g, ei0_g, ei1_s, e_pad, pcv)
    out = _tc_post_final(
        agg_lo2, agg_hi2, vemb,
        pcv["ln_post_g"].reshape(1, EMB), pcv["ln_post_b"].reshape(1, EMB),
        pcv["Wo1"].T, pcv["bo1"].reshape(1, EMB),
        pcv["Wo2"].T, pcv["bo2"].reshape(1, EMB),
        p["ln_v_g"].reshape(1, EMB), p["ln_v_b"].reshape(1, EMB),
        p["Wout1"].T, p["bout1"].reshape(1, EMB), p["Wout2"].T, blk)
    return out[:, 0]
